# Initial kernel scaffold; baseline (speedup 1.0000x reference)
import jax
import jax.numpy as jnp
from jax.experimental import pallas as pl


def kernel(position_sequence, params, particle_types, senders, receivers, n_particles_per_example):
    raise NotImplementedError("write your pallas kernel here")


# speedup vs baseline: 1.9873x; 1.9873x over previous
import functools

import jax
import jax.numpy as jnp
from jax import lax
from jax.experimental import pallas as pl
from jax.experimental.pallas import tpu as pltpu
from jax.experimental.pallas import tpu_sc as plsc

N = 10000
E = 160000
DIMS = 3
LATENT = 64
MP_STEPS = 5
RADIUS = 0.015
NUM_TYPES = 9
TYPE_EMB = 16

NPAD = 10240
EPAD = 163840
NC = 2
NS = 16
NW = NC * NS
CHK = 128
GCH = (2 * EPAD) // (NW * CHK)
SCH = EPAD // (NW * CHK)
NSUB = NPAD // NS

BN = 2048
BE = 2048

_f32 = jnp.float32


def _dot(x, w):
    return jax.lax.dot_general(x, w, (((1,), (0,)), ((), ())),
                               preferred_element_type=_f32)


def _ln(x):
    m = jnp.mean(x, axis=-1, keepdims=True)
    xc = x - m
    v = jnp.mean(xc * xc, axis=-1, keepdims=True)
    return xc * lax.rsqrt(v + 1e-6)


def _sc_gather(table, idx2d, ncols):
    nrows = idx2d.shape[0] * CHK
    nch = nrows // (NW * CHK)

    @functools.partial(
        pl.kernel,
        out_type=jax.ShapeDtypeStruct((nrows, ncols), _f32),
        mesh=plsc.VectorSubcoreMesh(core_axis_name="c", subcore_axis_name="s",
                                num_cores=NC, num_subcores=NS),
        scratch_types=[
            pltpu.VMEM((CHK,), jnp.int32),
            pltpu.VMEM((CHK, ncols), _f32),
            pltpu.SemaphoreType.DMA,
        ],
        compiler_params=pltpu.CompilerParams(use_tc_tiling_on_sc=False),
    )
    def k(table_hbm, idx_hbm, out_hbm, idx_v, rows_v, sem):
        w = lax.axis_index("s") * NC + lax.axis_index("c")

        @pl.loop(0, nch)
        def _(j):
            r = w * nch + j
            pltpu.sync_copy(idx_hbm.at[r], idx_v)
            pltpu.async_copy(table_hbm.at[idx_v], rows_v, sem).wait()
            pltpu.sync_copy(rows_v, out_hbm.at[pl.ds(r * CHK, CHK)])

    return k(table, idx2d)


def _sc_scatter(vals, idx2d, zeros_n):
    @functools.partial(
        pl.kernel,
        out_type=jax.ShapeDtypeStruct((NC, NPAD, LATENT), _f32),
        mesh=plsc.VectorSubcoreMesh(core_axis_name="c", subcore_axis_name="s",
                                    num_cores=NC, num_subcores=NS),
        scratch_types=[
            pltpu.VMEM_SHARED((NPAD, LATENT), _f32),
            pltpu.VMEM((SCH, CHK), jnp.int32),
            pltpu.VMEM((CHK, LATENT), _f32),
        ],
        compiler_params=pltpu.CompilerParams(use_tc_tiling_on_sc=False),
    )
    def k(vals_hbm, idx_hbm, zero_hbm, out_hbm, acc_sh, idx_v, vals_v):
        c = lax.axis_index("c")
        s = lax.axis_index("s")
        w = c * NS + s
        pltpu.sync_copy(zero_hbm.at[pl.ds(s * NSUB, NSUB)],
                        acc_sh.at[pl.ds(s * NSUB, NSUB)])
        pltpu.sync_copy(idx_hbm.at[pl.ds(w * SCH, SCH)], idx_v)
        plsc.subcore_barrier()

        @pl.loop(0, SCH)
        def _(j):
            pltpu.sync_copy(vals_hbm.at[pl.ds((w * SCH + j) * CHK, CHK)],
                            vals_v)
            pltpu.sync_copy(vals_v, acc_sh.at[idx_v.at[j]], add=True)

        plsc.subcore_barrier()
        pltpu.sync_copy(acc_sh.at[pl.ds(s * NSUB, NSUB)],
                        out_hbm.at[c].at[pl.ds(s * NSUB, NSUB)])

    return k(vals, idx2d, zeros_n)


def _enc_node_body(pos_ref, typ_ref, emb_ref, w1_ref, b1_ref, w2_ref, b2_ref,
                   ws_ref, wr_ref, v_ref, p_ref, pos8_ref):
    pos = pos_ref[...]
    nvel = pos[:, 3:18] - pos[:, 0:15]
    mr = pos[:, 15:18]
    dist = jnp.concatenate([mr - 0.1, 0.9 - mr], axis=1)
    distc = jnp.clip(dist * (1.0 / RADIUS), -1.0, 1.0)
    typ = typ_ref[...]
    oh = (typ == lax.broadcasted_iota(jnp.int32, (typ.shape[0], NUM_TYPES), 1))
    te = _dot(oh.astype(_f32), emb_ref[...])
    nf = jnp.concatenate([nvel, distc, te], axis=1)
    h = jnp.maximum(_dot(nf, w1_ref[...]) + b1_ref[...], 0.0)
    v = _ln(_dot(h, w2_ref[...]) + b2_ref[...])
    v_ref[...] = v
    p_ref[0] = _dot(v, ws_ref[...])
    p_ref[1] = _dot(v, wr_ref[...])
    pos8_ref[...] = jnp.concatenate(
        [mr, jnp.zeros((mr.shape[0], 5), _f32)], axis=1)


def _enc_edge_body(gs_ref, gr_ref, w1_ref, b1_ref, w2_ref, b2_ref, e_ref):
    rel = (gs_ref[:, 0:3] - gr_ref[:, 0:3]) * (1.0 / RADIUS)
    nrm = jnp.sqrt(jnp.sum(rel * rel, axis=1, keepdims=True))
    ef = jnp.concatenate([rel, nrm], axis=1)
    h = jnp.maximum(_dot(ef, w1_ref[...]) + b1_ref[...], 0.0)
    e_ref[...] = _ln(_dot(h, w2_ref[...]) + b2_ref[...])


def _edge_step_body(e_ref, gs_ref, gr_ref, w1e_ref, b1_ref, w2_ref, b2_ref,
                    enew_ref, eout_ref):
    e = e_ref[...]
    h = jnp.maximum(_dot(e, w1e_ref[...]) + gs_ref[...] + gr_ref[...]
                    + b1_ref[...], 0.0)
    en = _ln(_dot(h, w2_ref[...]) + b2_ref[...])
    enew_ref[...] = en
    eout_ref[...] = e + en


def _node_step_body(v_ref, parts_ref, wv_ref, wa_ref, b1_ref, w2_ref, b2_ref,
                    ws_ref, wr_ref, vout_ref, p_ref):
    v = v_ref[...]
    agg = parts_ref[0] + parts_ref[1]
    t = jnp.maximum(_dot(v, wv_ref[...]) + _dot(agg, wa_ref[...])
                    + b1_ref[...], 0.0)
    vo = v + _ln(_dot(t, w2_ref[...]) + b2_ref[...])
    vout_ref[...] = vo
    p_ref[0] = _dot(vo, ws_ref[...])
    p_ref[1] = _dot(vo, wr_ref[...])


def _node_last_body(v_ref, parts_ref, wv_ref, wa_ref, b1_ref, w2_ref, b2_ref,
                    vout_ref):
    v = v_ref[...]
    agg = parts_ref[0] + parts_ref[1]
    t = jnp.maximum(_dot(v, wv_ref[...]) + _dot(agg, wa_ref[...])
                    + b1_ref[...], 0.0)
    vout_ref[...] = v + _ln(_dot(t, w2_ref[...]) + b2_ref[...])


def _dec_body(v_ref, pos_ref, w1_ref, b1_ref, w2_ref, b2_ref, out_ref):
    v = v_ref[...]
    t = jnp.maximum(_dot(v, w1_ref[...]) + b1_ref[...], 0.0)
    acc = _dot(t, w2_ref[...]) + b2_ref[...]
    pos = pos_ref[...]
    mr = pos[:, 15:18]
    pv = pos[:, 12:15]
    out3 = mr + (mr - pv) + acc[:, 0:3]
    out_ref[...] = jnp.concatenate([out3, acc[:, 3:8]], axis=1)


def _wspec(shape):
    return pl.BlockSpec(shape, lambda i: tuple(0 for _ in shape))


def _enc_node_call(pos18, typ, emb, w1, b1, w2, b2, ws, wr):
    return pl.pallas_call(
        _enc_node_body,
        grid=(NPAD // BN,),
        in_specs=[
            pl.BlockSpec((BN, 18), lambda i: (i, 0)),
            pl.BlockSpec((BN, 1), lambda i: (i, 0)),
            _wspec((NUM_TYPES, TYPE_EMB)),
            _wspec((37, LATENT)), _wspec((1, LATENT)),
            _wspec((LATENT, LATENT)), _wspec((1, LATENT)),
            _wspec((LATENT, LATENT)), _wspec((LATENT, LATENT)),
        ],
        out_specs=[
            pl.BlockSpec((BN, LATENT), lambda i: (i, 0)),
            pl.BlockSpec((2, BN, LATENT), lambda i: (0, i, 0)),
            pl.BlockSpec((BN, 8), lambda i: (i, 0)),
        ],
        out_shape=[
            jax.ShapeDtypeStruct((NPAD, LATENT), _f32),
            jax.ShapeDtypeStruct((2, NPAD, LATENT), _f32),
            jax.ShapeDtypeStruct((NPAD, 8), _f32),
        ],
    )(pos18, typ, emb, w1, b1, w2, b2, ws, wr)


def _enc_edge_call(gpos, w1, b1, w2, b2):
    return pl.pallas_call(
        _enc_edge_body,
        grid=(EPAD // BE,),
        in_specs=[
            pl.BlockSpec((BE, 8), lambda i: (i, 0)),
            pl.BlockSpec((BE, 8), lambda i: (i + EPAD // BE, 0)),
            _wspec((4, LATENT)), _wspec((1, LATENT)),
            _wspec((LATENT, LATENT)), _wspec((1, LATENT)),
        ],
        out_specs=pl.BlockSpec((BE, LATENT), lambda i: (i, 0)),
        out_shape=jax.ShapeDtypeStruct((EPAD, LATENT), _f32),
    )(gpos, gpos, w1, b1, w2, b2)


def _edge_step_call(e, g, w1e, b1, w2, b2):
    return pl.pallas_call(
        _edge_step_body,
        grid=(EPAD // BE,),
        in_specs=[
            pl.BlockSpec((BE, LATENT), lambda i: (i, 0)),
            pl.BlockSpec((BE, LATENT), lambda i: (i, 0)),
            pl.BlockSpec((BE, LATENT), lambda i: (i + EPAD // BE, 0)),
            _wspec((LATENT, LATENT)), _wspec((1, LATENT)),
            _wspec((LATENT, LATENT)), _wspec((1, LATENT)),
        ],
        out_specs=[
            pl.BlockSpec((BE, LATENT), lambda i: (i, 0)),
            pl.BlockSpec((BE, LATENT), lambda i: (i, 0)),
        ],
        out_shape=[
            jax.ShapeDtypeStruct((EPAD, LATENT), _f32),
            jax.ShapeDtypeStruct((EPAD, LATENT), _f32),
        ],
    )(e, g, g, w1e, b1, w2, b2)


def _node_step_call(v, parts, wv, wa, b1, w2, b2, ws, wr):
    return pl.pallas_call(
        _node_step_body,
        grid=(NPAD // BN,),
        in_specs=[
            pl.BlockSpec((BN, LATENT), lambda i: (i, 0)),
            pl.BlockSpec((2, BN, LATENT), lambda i: (0, i, 0)),
            _wspec((LATENT, LATENT)), _wspec((LATENT, LATENT)),
            _wspec((1, LATENT)),
            _wspec((LATENT, LATENT)), _wspec((1, LATENT)),
            _wspec((LATENT, LATENT)), _wspec((LATENT, LATENT)),
        ],
        out_specs=[
            pl.BlockSpec((BN, LATENT), lambda i: (i, 0)),
            pl.BlockSpec((2, BN, LATENT), lambda i: (0, i, 0)),
        ],
        out_shape=[
            jax.ShapeDtypeStruct((NPAD, LATENT), _f32),
            jax.ShapeDtypeStruct((2, NPAD, LATENT), _f32),
        ],
    )(v, parts, wv, wa, b1, w2, b2, ws, wr)


def _node_last_call(v, parts, wv, wa, b1, w2, b2):
    return pl.pallas_call(
        _node_last_body,
        grid=(NPAD // BN,),
        in_specs=[
            pl.BlockSpec((BN, LATENT), lambda i: (i, 0)),
            pl.BlockSpec((2, BN, LATENT), lambda i: (0, i, 0)),
            _wspec((LATENT, LATENT)), _wspec((LATENT, LATENT)),
            _wspec((1, LATENT)),
            _wspec((LATENT, LATENT)), _wspec((1, LATENT)),
        ],
        out_specs=pl.BlockSpec((BN, LATENT), lambda i: (i, 0)),
        out_shape=jax.ShapeDtypeStruct((NPAD, LATENT), _f32),
    )(v, parts, wv, wa, b1, w2, b2)


def _dec_call(v, pos18, w1, b1, w2, b2):
    return pl.pallas_call(
        _dec_body,
        grid=(NPAD // BN,),
        in_specs=[
            pl.BlockSpec((BN, LATENT), lambda i: (i, 0)),
            pl.BlockSpec((BN, 18), lambda i: (i, 0)),
            _wspec((LATENT, LATENT)), _wspec((1, LATENT)),
            _wspec((LATENT, 8)), _wspec((1, 8)),
        ],
        out_specs=pl.BlockSpec((BN, 8), lambda i: (i, 0)),
        out_shape=jax.ShapeDtypeStruct((NPAD, 8), _f32),
    )(v, pos18, w1, b1, w2, b2)


def kernel(position_sequence, params, particle_types, senders, receivers,
           n_particles_per_example):
    pos18 = jnp.pad(position_sequence.reshape(N, 18), ((0, NPAD - N), (0, 0)))
    typ = jnp.pad(particle_types.astype(jnp.int32), (0, NPAD - N))
    typ = typ.reshape(NPAD, 1)

    npad_e = EPAD - E
    pad_spread = (jnp.arange(npad_e, dtype=jnp.int32) * 37) % N
    s_pad = jnp.concatenate([senders.astype(jnp.int32), pad_spread])
    r_pad = jnp.concatenate([receivers.astype(jnp.int32), pad_spread])
    idx_gather = jnp.concatenate([s_pad, r_pad + NPAD]).reshape(-1, CHK)
    pad_sink = N + (jnp.arange(npad_e, dtype=jnp.int32) % (NPAD - N))
    ridx = jnp.concatenate(
        [receivers.astype(jnp.int32), pad_sink]).reshape(-1, CHK)
    zeros_n = jnp.zeros((NPAD, LATENT), _f32)

    def b2d(b):
        return b.reshape(1, -1)

    (wn1, bn1), (wn2, bn2) = params['enc_node']
    (we1, be1), (we2, be2) = params['enc_edge']
    (wd1, bd1), (wd2, bd2) = params['dec']
    wd2p = jnp.pad(wd2, ((0, 0), (0, 8 - DIMS)))
    bd2p = jnp.pad(bd2, (0, 8 - DIMS))

    ew1 = [sp['edge'][0][0] for sp in params['proc']]
    ws_all = [w[LATENT:2 * LATENT] for w in ew1]
    wr_all = [w[2 * LATENT:] for w in ew1]

    v, p, pos8 = _enc_node_call(pos18, typ, params['type_emb'],
                                wn1, b2d(bn1), wn2, b2d(bn2),
                                ws_all[0], wr_all[0])

    gpos = _sc_gather(jnp.concatenate([pos8, pos8], axis=0), idx_gather, 8)
    e = _enc_edge_call(gpos, we1, b2d(be1), we2, b2d(be2))

    for step in range(MP_STEPS):
        sp = params['proc'][step]
        (w1, b1), (w2, b2) = sp['edge']
        g = _sc_gather(p.reshape(2 * NPAD, LATENT), idx_gather, LATENT)
        e_new, e = _edge_step_call(e, g, w1[:LATENT], b2d(b1), w2, b2d(b2))
        parts = _sc_scatter(e_new, ridx, zeros_n)
        (nw1, nb1), (nw2, nb2) = sp['node']
        wv, wa = nw1[:LATENT], nw1[LATENT:]
        if step < MP_STEPS - 1:
            v, p = _node_step_call(v, parts, wv, wa, b2d(nb1), nw2, b2d(nb2),
                                   ws_all[step + 1], wr_all[step + 1])
        else:
            v = _node_last_call(v, parts, wv, wa, b2d(nb1), nw2, b2d(nb2))

    out = _dec_call(v, pos18, wd1, b2d(bd1), wd2p, b2d(bd2p))
    return out[:N, :DIMS]


# speedup vs baseline: 2.4199x; 1.2177x over previous
import functools

import jax
import jax.numpy as jnp
from jax import lax
from jax.experimental import pallas as pl
from jax.experimental.pallas import tpu as pltpu
from jax.experimental.pallas import tpu_sc as plsc

N = 10000
E = 160000
DIMS = 3
LATENT = 64
MP_STEPS = 5
RADIUS = 0.015
NUM_TYPES = 9
TYPE_EMB = 16

NPAD = 10240
EPAD = 163840
NC = 2
NS = 16
NW = NC * NS
CHK = 128
GCH = (2 * EPAD) // (NW * CHK)
SCH = EPAD // (NW * CHK)
NSUB = NPAD // NS

BN = 2048
BE = 2048

_f32 = jnp.float32


def _dot(x, w):
    return jax.lax.dot_general(x, w, (((1,), (0,)), ((), ())),
                               preferred_element_type=_f32)


def _ln(x):
    m = jnp.mean(x, axis=-1, keepdims=True)
    xc = x - m
    v = jnp.mean(xc * xc, axis=-1, keepdims=True)
    return xc * lax.rsqrt(v + 1e-6)


NB = 4


def _sc_gather(table, idx2d, ncols):
    nrows = idx2d.shape[0] * CHK
    nch = nrows // (NW * CHK)
    ngr = nch // NB

    @functools.partial(
        pl.kernel,
        out_type=jax.ShapeDtypeStruct((nrows, ncols), _f32),
        mesh=plsc.VectorSubcoreMesh(core_axis_name="c", subcore_axis_name="s",
                                    num_cores=NC, num_subcores=NS),
        scratch_types=[
            pltpu.VMEM((NB, CHK), jnp.int32),
            pltpu.VMEM((NB * CHK, ncols), _f32),
            [pltpu.SemaphoreType.DMA] * NB,
            [pltpu.SemaphoreType.DMA] * NB,
            pltpu.SemaphoreType.DMA,
        ],
        compiler_params=pltpu.CompilerParams(use_tc_tiling_on_sc=False),
    )
    def k(table_hbm, idx_hbm, out_hbm, idx_v, rows_v, semi, semg, semo):
        w = lax.axis_index("s") * NC + lax.axis_index("c")
        base = w * nch

        for b in range(NB):
            pltpu.async_copy(idx_hbm.at[base + b], idx_v.at[b], semi[b])

        @pl.loop(0, ngr)
        def _(g):
            gbase = base + g * NB

            @pl.when(g > 0)
            def _():
                pltpu.make_async_copy(
                    rows_v, out_hbm.at[pl.ds(gbase * CHK, NB * CHK)],
                    semo).wait()

            for b in range(NB):
                pltpu.make_async_copy(idx_hbm.at[gbase + b], idx_v.at[b],
                                      semi[b]).wait()
                pltpu.async_copy(table_hbm.at[idx_v.at[b]],
                                 rows_v.at[pl.ds(b * CHK, CHK)], semg[b])
            for b in range(NB):
                pltpu.make_async_copy(table_hbm.at[idx_v.at[b]],
                                      rows_v.at[pl.ds(b * CHK, CHK)],
                                      semg[b]).wait()

                @pl.when(g + 1 < ngr)
                def _():
                    pltpu.async_copy(idx_hbm.at[gbase + NB + b], idx_v.at[b],
                                     semi[b])

            pltpu.async_copy(rows_v, out_hbm.at[pl.ds(gbase * CHK, NB * CHK)],
                             semo)

        pltpu.make_async_copy(
            rows_v, out_hbm.at[pl.ds((base + nch - NB) * CHK, NB * CHK)],
            semo).wait()

    return k(table, idx2d)


def _sc_scatter(vals, idx2d, zeros_n):
    @functools.partial(
        pl.kernel,
        out_type=jax.ShapeDtypeStruct((NC, NPAD, LATENT), _f32),
        mesh=plsc.VectorSubcoreMesh(core_axis_name="c", subcore_axis_name="s",
                                    num_cores=NC, num_subcores=NS),
        scratch_types=[
            pltpu.VMEM_SHARED((NPAD, LATENT), _f32),
            pltpu.VMEM((SCH, CHK), jnp.int32),
            pltpu.VMEM((NB, CHK, LATENT), _f32),
            [pltpu.SemaphoreType.DMA] * NB,
            [pltpu.SemaphoreType.DMA] * NB,
        ],
        compiler_params=pltpu.CompilerParams(use_tc_tiling_on_sc=False),
    )
    def k(vals_hbm, idx_hbm, zero_hbm, out_hbm, acc_sh, idx_v, vals_v,
          seml, sems):
        c = lax.axis_index("c")
        s = lax.axis_index("s")
        w = c * NS + s
        pltpu.sync_copy(zero_hbm.at[pl.ds(s * NSUB, NSUB)],
                        acc_sh.at[pl.ds(s * NSUB, NSUB)])
        pltpu.sync_copy(idx_hbm.at[pl.ds(w * SCH, SCH)], idx_v)
        for b in range(NB):
            pltpu.async_copy(
                vals_hbm.at[pl.ds((w * SCH + b) * CHK, CHK)],
                vals_v.at[b], seml[b])
        plsc.subcore_barrier()

        @pl.loop(0, SCH // NB)
        def _(g):
            jbase = w * SCH + g * NB
            for b in range(NB):
                pltpu.make_async_copy(
                    vals_hbm.at[pl.ds((jbase + b) * CHK, CHK)],
                    vals_v.at[b], seml[b]).wait()
                pltpu.async_copy(vals_v.at[b],
                                 acc_sh.at[idx_v.at[g * NB + b]],
                                 sems[b], add=True)
            for b in range(NB):
                pltpu.make_async_copy(vals_v.at[b],
                                      acc_sh.at[idx_v.at[g * NB + b]],
                                      sems[b]).wait()

                @pl.when(g + 1 < SCH // NB)
                def _():
                    pltpu.async_copy(
                        vals_hbm.at[pl.ds((jbase + NB + b) * CHK, CHK)],
                        vals_v.at[b], seml[b])

        plsc.subcore_barrier()
        pltpu.sync_copy(acc_sh.at[pl.ds(s * NSUB, NSUB)],
                        out_hbm.at[c].at[pl.ds(s * NSUB, NSUB)])

    return k(vals, idx2d, zeros_n)


def _enc_node_body(pos_ref, typ_ref, emb_ref, w1_ref, b1_ref, w2_ref, b2_ref,
                   ws_ref, wr_ref, v_ref, p_ref, pos8_ref):
    pos = pos_ref[...]
    nvel = pos[:, 3:18] - pos[:, 0:15]
    mr = pos[:, 15:18]
    dist = jnp.concatenate([mr - 0.1, 0.9 - mr], axis=1)
    distc = jnp.clip(dist * (1.0 / RADIUS), -1.0, 1.0)
    typ = typ_ref[...]
    oh = (typ == lax.broadcasted_iota(jnp.int32, (typ.shape[0], NUM_TYPES), 1))
    te = _dot(oh.astype(_f32), emb_ref[...])
    nf = jnp.concatenate([nvel, distc, te], axis=1)
    h = jnp.maximum(_dot(nf, w1_ref[...]) + b1_ref[...], 0.0)
    v = _ln(_dot(h, w2_ref[...]) + b2_ref[...])
    v_ref[...] = v
    p_ref[0] = _dot(v, ws_ref[...])
    p_ref[1] = _dot(v, wr_ref[...])
    pos8_ref[...] = jnp.concatenate(
        [mr, jnp.zeros((mr.shape[0], 5), _f32)], axis=1)


def _enc_edge_body(gs_ref, gr_ref, w1_ref, b1_ref, w2_ref, b2_ref, e_ref):
    rel = (gs_ref[:, 0:3] - gr_ref[:, 0:3]) * (1.0 / RADIUS)
    nrm = jnp.sqrt(jnp.sum(rel * rel, axis=1, keepdims=True))
    ef = jnp.concatenate([rel, nrm], axis=1)
    h = jnp.maximum(_dot(ef, w1_ref[...]) + b1_ref[...], 0.0)
    e_ref[...] = _ln(_dot(h, w2_ref[...]) + b2_ref[...])


def _edge_step_body(e_ref, gs_ref, gr_ref, w1e_ref, b1_ref, w2_ref, b2_ref,
                    enew_ref, eout_ref):
    e = e_ref[...]
    h = jnp.maximum(_dot(e, w1e_ref[...]) + gs_ref[...] + gr_ref[...]
                    + b1_ref[...], 0.0)
    en = _ln(_dot(h, w2_ref[...]) + b2_ref[...])
    enew_ref[...] = en
    eout_ref[...] = e + en


def _node_step_body(v_ref, parts_ref, wv_ref, wa_ref, b1_ref, w2_ref, b2_ref,
                    ws_ref, wr_ref, vout_ref, p_ref):
    v = v_ref[...]
    agg = parts_ref[0] + parts_ref[1]
    t = jnp.maximum(_dot(v, wv_ref[...]) + _dot(agg, wa_ref[...])
                    + b1_ref[...], 0.0)
    vo = v + _ln(_dot(t, w2_ref[...]) + b2_ref[...])
    vout_ref[...] = vo
    p_ref[0] = _dot(vo, ws_ref[...])
    p_ref[1] = _dot(vo, wr_ref[...])


def _node_last_body(v_ref, parts_ref, wv_ref, wa_ref, b1_ref, w2_ref, b2_ref,
                    vout_ref):
    v = v_ref[...]
    agg = parts_ref[0] + parts_ref[1]
    t = jnp.maximum(_dot(v, wv_ref[...]) + _dot(agg, wa_ref[...])
                    + b1_ref[...], 0.0)
    vout_ref[...] = v + _ln(_dot(t, w2_ref[...]) + b2_ref[...])


def _dec_body(v_ref, pos_ref, w1_ref, b1_ref, w2_ref, b2_ref, out_ref):
    v = v_ref[...]
    t = jnp.maximum(_dot(v, w1_ref[...]) + b1_ref[...], 0.0)
    acc = _dot(t, w2_ref[...]) + b2_ref[...]
    pos = pos_ref[...]
    mr = pos[:, 15:18]
    pv = pos[:, 12:15]
    out3 = mr + (mr - pv) + acc[:, 0:3]
    out_ref[...] = jnp.concatenate([out3, acc[:, 3:8]], axis=1)


def _wspec(shape):
    return pl.BlockSpec(shape, lambda i: tuple(0 for _ in shape))


def _enc_node_call(pos18, typ, emb, w1, b1, w2, b2, ws, wr):
    return pl.pallas_call(
        _enc_node_body,
        grid=(NPAD // BN,),
        in_specs=[
            pl.BlockSpec((BN, 18), lambda i: (i, 0)),
            pl.BlockSpec((BN, 1), lambda i: (i, 0)),
            _wspec((NUM_TYPES, TYPE_EMB)),
            _wspec((37, LATENT)), _wspec((1, LATENT)),
            _wspec((LATENT, LATENT)), _wspec((1, LATENT)),
            _wspec((LATENT, LATENT)), _wspec((LATENT, LATENT)),
        ],
        out_specs=[
            pl.BlockSpec((BN, LATENT), lambda i: (i, 0)),
            pl.BlockSpec((2, BN, LATENT), lambda i: (0, i, 0)),
            pl.BlockSpec((BN, 8), lambda i: (i, 0)),
        ],
        out_shape=[
            jax.ShapeDtypeStruct((NPAD, LATENT), _f32),
            jax.ShapeDtypeStruct((2, NPAD, LATENT), _f32),
            jax.ShapeDtypeStruct((NPAD, 8), _f32),
        ],
    )(pos18, typ, emb, w1, b1, w2, b2, ws, wr)


def _enc_edge_call(gpos, w1, b1, w2, b2):
    return pl.pallas_call(
        _enc_edge_body,
        grid=(EPAD // BE,),
        in_specs=[
            pl.BlockSpec((BE, 8), lambda i: (i, 0)),
            pl.BlockSpec((BE, 8), lambda i: (i + EPAD // BE, 0)),
            _wspec((4, LATENT)), _wspec((1, LATENT)),
            _wspec((LATENT, LATENT)), _wspec((1, LATENT)),
        ],
        out_specs=pl.BlockSpec((BE, LATENT), lambda i: (i, 0)),
        out_shape=jax.ShapeDtypeStruct((EPAD, LATENT), _f32),
    )(gpos, gpos, w1, b1, w2, b2)


def _edge_step_call(e, g, w1e, b1, w2, b2):
    return pl.pallas_call(
        _edge_step_body,
        grid=(EPAD // BE,),
        in_specs=[
            pl.BlockSpec((BE, LATENT), lambda i: (i, 0)),
            pl.BlockSpec((BE, LATENT), lambda i: (i, 0)),
            pl.BlockSpec((BE, LATENT), lambda i: (i + EPAD // BE, 0)),
            _wspec((LATENT, LATENT)), _wspec((1, LATENT)),
            _wspec((LATENT, LATENT)), _wspec((1, LATENT)),
        ],
        out_specs=[
            pl.BlockSpec((BE, LATENT), lambda i: (i, 0)),
            pl.BlockSpec((BE, LATENT), lambda i: (i, 0)),
        ],
        out_shape=[
            jax.ShapeDtypeStruct((EPAD, LATENT), _f32),
            jax.ShapeDtypeStruct((EPAD, LATENT), _f32),
        ],
    )(e, g, g, w1e, b1, w2, b2)


def _node_step_call(v, parts, wv, wa, b1, w2, b2, ws, wr):
    return pl.pallas_call(
        _node_step_body,
        grid=(NPAD // BN,),
        in_specs=[
            pl.BlockSpec((BN, LATENT), lambda i: (i, 0)),
            pl.BlockSpec((2, BN, LATENT), lambda i: (0, i, 0)),
            _wspec((LATENT, LATENT)), _wspec((LATENT, LATENT)),
            _wspec((1, LATENT)),
            _wspec((LATENT, LATENT)), _wspec((1, LATENT)),
            _wspec((LATENT, LATENT)), _wspec((LATENT, LATENT)),
        ],
        out_specs=[
            pl.BlockSpec((BN, LATENT), lambda i: (i, 0)),
            pl.BlockSpec((2, BN, LATENT), lambda i: (0, i, 0)),
        ],
        out_shape=[
            jax.ShapeDtypeStruct((NPAD, LATENT), _f32),
            jax.ShapeDtypeStruct((2, NPAD, LATENT), _f32),
        ],
    )(v, parts, wv, wa, b1, w2, b2, ws, wr)


def _node_last_call(v, parts, wv, wa, b1, w2, b2):
    return pl.pallas_call(
        _node_last_body,
        grid=(NPAD // BN,),
        in_specs=[
            pl.BlockSpec((BN, LATENT), lambda i: (i, 0)),
            pl.BlockSpec((2, BN, LATENT), lambda i: (0, i, 0)),
            _wspec((LATENT, LATENT)), _wspec((LATENT, LATENT)),
            _wspec((1, LATENT)),
            _wspec((LATENT, LATENT)), _wspec((1, LATENT)),
        ],
        out_specs=pl.BlockSpec((BN, LATENT), lambda i: (i, 0)),
        out_shape=jax.ShapeDtypeStruct((NPAD, LATENT), _f32),
    )(v, parts, wv, wa, b1, w2, b2)


def _dec_call(v, pos18, w1, b1, w2, b2):
    return pl.pallas_call(
        _dec_body,
        grid=(NPAD // BN,),
        in_specs=[
            pl.BlockSpec((BN, LATENT), lambda i: (i, 0)),
            pl.BlockSpec((BN, 18), lambda i: (i, 0)),
            _wspec((LATENT, LATENT)), _wspec((1, LATENT)),
            _wspec((LATENT, 8)), _wspec((1, 8)),
        ],
        out_specs=pl.BlockSpec((BN, 8), lambda i: (i, 0)),
        out_shape=jax.ShapeDtypeStruct((NPAD, 8), _f32),
    )(v, pos18, w1, b1, w2, b2)


def kernel(position_sequence, params, particle_types, senders, receivers,
           n_particles_per_example):
    pos18 = jnp.pad(position_sequence.reshape(N, 18), ((0, NPAD - N), (0, 0)))
    typ = jnp.pad(particle_types.astype(jnp.int32), (0, NPAD - N))
    typ = typ.reshape(NPAD, 1)

    npad_e = EPAD - E
    pad_spread = (jnp.arange(npad_e, dtype=jnp.int32) * 37) % N
    s_pad = jnp.concatenate([senders.astype(jnp.int32), pad_spread])
    r_pad = jnp.concatenate([receivers.astype(jnp.int32), pad_spread])
    idx_gather = jnp.concatenate([s_pad, r_pad + NPAD]).reshape(-1, CHK)
    pad_sink = N + (jnp.arange(npad_e, dtype=jnp.int32) % (NPAD - N))
    ridx = jnp.concatenate(
        [receivers.astype(jnp.int32), pad_sink]).reshape(-1, CHK)
    zeros_n = jnp.zeros((NPAD, LATENT), _f32)

    def b2d(b):
        return b.reshape(1, -1)

    (wn1, bn1), (wn2, bn2) = params['enc_node']
    (we1, be1), (we2, be2) = params['enc_edge']
    (wd1, bd1), (wd2, bd2) = params['dec']
    wd2p = jnp.pad(wd2, ((0, 0), (0, 8 - DIMS)))
    bd2p = jnp.pad(bd2, (0, 8 - DIMS))

    ew1 = [sp['edge'][0][0] for sp in params['proc']]
    ws_all = [w[LATENT:2 * LATENT] for w in ew1]
    wr_all = [w[2 * LATENT:] for w in ew1]

    v, p, pos8 = _enc_node_call(pos18, typ, params['type_emb'],
                                wn1, b2d(bn1), wn2, b2d(bn2),
                                ws_all[0], wr_all[0])

    gpos = _sc_gather(jnp.concatenate([pos8, pos8], axis=0), idx_gather, 8)
    e = _enc_edge_call(gpos, we1, b2d(be1), we2, b2d(be2))

    for step in range(MP_STEPS):
        sp = params['proc'][step]
        (w1, b1), (w2, b2) = sp['edge']
        g = _sc_gather(p.reshape(2 * NPAD, LATENT), idx_gather, LATENT)
        e_new, e = _edge_step_call(e, g, w1[:LATENT], b2d(b1), w2, b2d(b2))
        parts = _sc_scatter(e_new, ridx, zeros_n)
        (nw1, nb1), (nw2, nb2) = sp['node']
        wv, wa = nw1[:LATENT], nw1[LATENT:]
        if step < MP_STEPS - 1:
            v, p = _node_step_call(v, parts, wv, wa, b2d(nb1), nw2, b2d(nb2),
                                   ws_all[step + 1], wr_all[step + 1])
        else:
            v = _node_last_call(v, parts, wv, wa, b2d(nb1), nw2, b2d(nb2))

    out = _dec_call(v, pos18, wd1, b2d(bd1), wd2p, b2d(bd2p))
    return out[:N, :DIMS]


# speedup vs baseline: 3.0112x; 1.2443x over previous
import functools

import jax
import jax.numpy as jnp
from jax import lax
from jax.experimental import pallas as pl
from jax.experimental.pallas import tpu as pltpu
from jax.experimental.pallas import tpu_sc as plsc

N = 10000
E = 160000
DIMS = 3
LATENT = 64
MP_STEPS = 5
RADIUS = 0.015
NUM_TYPES = 9
TYPE_EMB = 16

NPAD = 10240
EPAD = 163840
NC = 2
NS = 16
NW = NC * NS
CHK = 128
GCH = (2 * EPAD) // (NW * CHK)
SCH = EPAD // (NW * CHK)
NSUB = NPAD // NS

BN = 2048
BE = 2048

_f32 = jnp.float32


def _dot(x, w):
    return jax.lax.dot_general(x, w, (((1,), (0,)), ((), ())),
                               preferred_element_type=_f32)


def _ln(x):
    m = jnp.mean(x, axis=-1, keepdims=True)
    xc = x - m
    v = jnp.mean(xc * xc, axis=-1, keepdims=True)
    return xc * lax.rsqrt(v + 1e-6)


NB = 4
NBS = 2


def _sc_gather(table, idx2d):
    ncols = 128
    nrows = idx2d.shape[0] * CHK
    nch = nrows // (NW * CHK)
    ngr = nch // NB

    @functools.partial(
        pl.kernel,
        out_type=jax.ShapeDtypeStruct((nrows, ncols), _f32),
        mesh=plsc.VectorSubcoreMesh(core_axis_name="c", subcore_axis_name="s",
                                    num_cores=NC, num_subcores=NS),
        scratch_types=[
            pltpu.VMEM((NB, CHK), jnp.int32),
            pltpu.VMEM((NB * CHK, ncols), _f32),
            [pltpu.SemaphoreType.DMA] * NB,
            [pltpu.SemaphoreType.DMA] * NB,
            pltpu.SemaphoreType.DMA,
        ],
    )
    def k(table_hbm, idx_hbm, out_hbm, idx_v, rows_v, semi, semg, semo):
        w = lax.axis_index("s") * NC + lax.axis_index("c")
        base = w * nch

        for b in range(NB):
            pltpu.async_copy(idx_hbm.at[base + b], idx_v.at[b], semi[b])

        @pl.loop(0, ngr)
        def _(g):
            gbase = base + g * NB

            @pl.when(g > 0)
            def _():
                pltpu.make_async_copy(
                    rows_v, out_hbm.at[pl.ds(gbase * CHK, NB * CHK)],
                    semo).wait()

            for b in range(NB):
                pltpu.make_async_copy(idx_hbm.at[gbase + b], idx_v.at[b],
                                      semi[b]).wait()
                pltpu.async_copy(table_hbm.at[idx_v.at[b]],
                                 rows_v.at[pl.ds(b * CHK, CHK)], semg[b])
            for b in range(NB):
                pltpu.make_async_copy(table_hbm.at[idx_v.at[b]],
                                      rows_v.at[pl.ds(b * CHK, CHK)],
                                      semg[b]).wait()

                @pl.when(g + 1 < ngr)
                def _():
                    pltpu.async_copy(idx_hbm.at[gbase + NB + b], idx_v.at[b],
                                     semi[b])

            pltpu.async_copy(rows_v, out_hbm.at[pl.ds(gbase * CHK, NB * CHK)],
                             semo)

        pltpu.make_async_copy(
            rows_v, out_hbm.at[pl.ds((base + nch - NB) * CHK, NB * CHK)],
            semo).wait()

    return k(table, idx2d)


def _sc_scatter(vals, idx2d, zeros_n):
    @functools.partial(
        pl.kernel,
        out_type=jax.ShapeDtypeStruct((NC, NPAD, 128), _f32),
        mesh=plsc.VectorSubcoreMesh(core_axis_name="c", subcore_axis_name="s",
                                    num_cores=NC, num_subcores=NS),
        scratch_types=[
            pltpu.VMEM_SHARED((NPAD, 128), _f32),
            pltpu.VMEM((SCH, CHK), jnp.int32),
            pltpu.VMEM((NBS, CHK, 128), _f32),
            [pltpu.SemaphoreType.DMA] * NBS,
            [pltpu.SemaphoreType.DMA] * NBS,
        ],
    )
    def k(vals_hbm, idx_hbm, zero_hbm, out_hbm, acc_sh, idx_v, vals_v,
          seml, sems):
        c = lax.axis_index("c")
        s = lax.axis_index("s")
        w = c * NS + s
        pltpu.sync_copy(zero_hbm.at[pl.ds(s * NSUB, NSUB)],
                        acc_sh.at[pl.ds(s * NSUB, NSUB)])
        pltpu.sync_copy(idx_hbm.at[pl.ds(w * SCH, SCH)], idx_v)
        for b in range(NBS):
            pltpu.async_copy(
                vals_hbm.at[pl.ds((w * SCH + b) * CHK, CHK)],
                vals_v.at[b], seml[b])
        plsc.subcore_barrier()

        @pl.loop(0, SCH // NBS)
        def _(g):
            jbase = w * SCH + g * NBS
            for b in range(NBS):
                pltpu.make_async_copy(
                    vals_hbm.at[pl.ds((jbase + b) * CHK, CHK)],
                    vals_v.at[b], seml[b]).wait()
                pltpu.async_copy(vals_v.at[b],
                                 acc_sh.at[idx_v.at[g * NBS + b]],
                                 sems[b], add=True)
            for b in range(NBS):
                pltpu.make_async_copy(vals_v.at[b],
                                      acc_sh.at[idx_v.at[g * NBS + b]],
                                      sems[b]).wait()

                @pl.when(g + 1 < SCH // NBS)
                def _():
                    pltpu.async_copy(
                        vals_hbm.at[pl.ds((jbase + NBS + b) * CHK, CHK)],
                        vals_v.at[b], seml[b])

        plsc.subcore_barrier()
        pltpu.sync_copy(acc_sh.at[pl.ds(s * NSUB, NSUB)],
                        out_hbm.at[c].at[pl.ds(s * NSUB, NSUB)])

    return k(vals, idx2d, zeros_n)


def _enc_node_body(pos_ref, typ_ref, emb_ref, w1_ref, b1_ref, w2_ref, b2_ref,
                   ws_ref, wr_ref, v_ref, p_ref, pos8_ref):
    pos = pos_ref[...]
    nvel = pos[:, 3:18] - pos[:, 0:15]
    mr = pos[:, 15:18]
    dist = jnp.concatenate([mr - 0.1, 0.9 - mr], axis=1)
    distc = jnp.clip(dist * (1.0 / RADIUS), -1.0, 1.0)
    typ = typ_ref[...]
    oh = (typ == lax.broadcasted_iota(jnp.int32, (typ.shape[0], NUM_TYPES), 1))
    te = _dot(oh.astype(_f32), emb_ref[...])
    nf = jnp.concatenate([nvel, distc, te], axis=1)
    h = jnp.maximum(_dot(nf, w1_ref[...]) + b1_ref[...], 0.0)
    v = _ln(_dot(h, w2_ref[...]) + b2_ref[...])
    v_ref[...] = v
    p_ref[...] = jnp.concatenate(
        [_dot(v, ws_ref[...]), _dot(v, wr_ref[...])], axis=1)
    z61 = jnp.zeros((mr.shape[0], 61), _f32)
    pos8_ref[...] = jnp.concatenate([mr, z61, mr, z61], axis=1)


def _enc_edge_body(gs_ref, gr_ref, w1_ref, b1_ref, w2_ref, b2_ref, e_ref):
    rel = (gs_ref[:, 0:3] - gr_ref[:, 64:67]) * (1.0 / RADIUS)
    nrm = jnp.sqrt(jnp.sum(rel * rel, axis=1, keepdims=True))
    ef = jnp.concatenate([rel, nrm], axis=1)
    h = jnp.maximum(_dot(ef, w1_ref[...]) + b1_ref[...], 0.0)
    e_ref[...] = _ln(_dot(h, w2_ref[...]) + b2_ref[...])


def _edge_step_body(e_ref, gs_ref, gr_ref, w1e_ref, b1_ref, w2_ref, b2_ref,
                    enew_ref, eout_ref):
    e = e_ref[...]
    h = jnp.maximum(_dot(e, w1e_ref[...]) + gs_ref[:, 0:LATENT]
                    + gr_ref[:, LATENT:] + b1_ref[...], 0.0)
    en = _ln(_dot(h, w2_ref[...]) + b2_ref[...])
    enew_ref[...] = jnp.concatenate(
        [en, jnp.zeros((en.shape[0], 128 - LATENT), _f32)], axis=1)
    eout_ref[...] = e + en


def _node_step_body(v_ref, parts_ref, wv_ref, wa_ref, b1_ref, w2_ref, b2_ref,
                    ws_ref, wr_ref, vout_ref, p_ref):
    v = v_ref[...]
    agg = parts_ref[0, :, 0:LATENT] + parts_ref[1, :, 0:LATENT]
    t = jnp.maximum(_dot(v, wv_ref[...]) + _dot(agg, wa_ref[...])
                    + b1_ref[...], 0.0)
    vo = v + _ln(_dot(t, w2_ref[...]) + b2_ref[...])
    vout_ref[...] = vo
    p_ref[...] = jnp.concatenate(
        [_dot(vo, ws_ref[...]), _dot(vo, wr_ref[...])], axis=1)


def _node_last_body(v_ref, parts_ref, wv_ref, wa_ref, b1_ref, w2_ref, b2_ref,
                    vout_ref):
    v = v_ref[...]
    agg = parts_ref[0, :, 0:LATENT] + parts_ref[1, :, 0:LATENT]
    t = jnp.maximum(_dot(v, wv_ref[...]) + _dot(agg, wa_ref[...])
                    + b1_ref[...], 0.0)
    vout_ref[...] = v + _ln(_dot(t, w2_ref[...]) + b2_ref[...])


def _dec_body(v_ref, pos_ref, w1_ref, b1_ref, w2_ref, b2_ref, out_ref):
    v = v_ref[...]
    t = jnp.maximum(_dot(v, w1_ref[...]) + b1_ref[...], 0.0)
    acc = _dot(t, w2_ref[...]) + b2_ref[...]
    pos = pos_ref[...]
    mr = pos[:, 15:18]
    pv = pos[:, 12:15]
    out3 = mr + (mr - pv) + acc[:, 0:3]
    out_ref[...] = jnp.concatenate([out3, acc[:, 3:8]], axis=1)


def _wspec(shape):
    return pl.BlockSpec(shape, lambda i: tuple(0 for _ in shape))


def _enc_node_call(pos18, typ, emb, w1, b1, w2, b2, ws, wr):
    return pl.pallas_call(
        _enc_node_body,
        grid=(NPAD // BN,),
        in_specs=[
            pl.BlockSpec((BN, 18), lambda i: (i, 0)),
            pl.BlockSpec((BN, 1), lambda i: (i, 0)),
            _wspec((NUM_TYPES, TYPE_EMB)),
            _wspec((37, LATENT)), _wspec((1, LATENT)),
            _wspec((LATENT, LATENT)), _wspec((1, LATENT)),
            _wspec((LATENT, LATENT)), _wspec((LATENT, LATENT)),
        ],
        out_specs=[
            pl.BlockSpec((BN, LATENT), lambda i: (i, 0)),
            pl.BlockSpec((BN, 128), lambda i: (i, 0)),
            pl.BlockSpec((BN, 128), lambda i: (i, 0)),
        ],
        out_shape=[
            jax.ShapeDtypeStruct((NPAD, LATENT), _f32),
            jax.ShapeDtypeStruct((NPAD, 128), _f32),
            jax.ShapeDtypeStruct((NPAD, 128), _f32),
        ],
    )(pos18, typ, emb, w1, b1, w2, b2, ws, wr)


def _enc_edge_call(gpos, w1, b1, w2, b2):
    return pl.pallas_call(
        _enc_edge_body,
        grid=(EPAD // BE,),
        in_specs=[
            pl.BlockSpec((BE, 128), lambda i: (i, 0)),
            pl.BlockSpec((BE, 128), lambda i: (i + EPAD // BE, 0)),
            _wspec((4, LATENT)), _wspec((1, LATENT)),
            _wspec((LATENT, LATENT)), _wspec((1, LATENT)),
        ],
        out_specs=pl.BlockSpec((BE, LATENT), lambda i: (i, 0)),
        out_shape=jax.ShapeDtypeStruct((EPAD, LATENT), _f32),
    )(gpos, gpos, w1, b1, w2, b2)


def _edge_step_call(e, g, w1e, b1, w2, b2):
    return pl.pallas_call(
        _edge_step_body,
        grid=(EPAD // BE,),
        in_specs=[
            pl.BlockSpec((BE, LATENT), lambda i: (i, 0)),
            pl.BlockSpec((BE, 128), lambda i: (i, 0)),
            pl.BlockSpec((BE, 128), lambda i: (i + EPAD // BE, 0)),
            _wspec((LATENT, LATENT)), _wspec((1, LATENT)),
            _wspec((LATENT, LATENT)), _wspec((1, LATENT)),
        ],
        out_specs=[
            pl.BlockSpec((BE, 128), lambda i: (i, 0)),
            pl.BlockSpec((BE, LATENT), lambda i: (i, 0)),
        ],
        out_shape=[
            jax.ShapeDtypeStruct((EPAD, 128), _f32),
            jax.ShapeDtypeStruct((EPAD, LATENT), _f32),
        ],
    )(e, g, g, w1e, b1, w2, b2)


def _node_step_call(v, parts, wv, wa, b1, w2, b2, ws, wr):
    return pl.pallas_call(
        _node_step_body,
        grid=(NPAD // BN,),
        in_specs=[
            pl.BlockSpec((BN, LATENT), lambda i: (i, 0)),
            pl.BlockSpec((2, BN, 128), lambda i: (0, i, 0)),
            _wspec((LATENT, LATENT)), _wspec((LATENT, LATENT)),
            _wspec((1, LATENT)),
            _wspec((LATENT, LATENT)), _wspec((1, LATENT)),
            _wspec((LATENT, LATENT)), _wspec((LATENT, LATENT)),
        ],
        out_specs=[
            pl.BlockSpec((BN, LATENT), lambda i: (i, 0)),
            pl.BlockSpec((BN, 128), lambda i: (i, 0)),
        ],
        out_shape=[
            jax.ShapeDtypeStruct((NPAD, LATENT), _f32),
            jax.ShapeDtypeStruct((NPAD, 128), _f32),
        ],
    )(v, parts, wv, wa, b1, w2, b2, ws, wr)


def _node_last_call(v, parts, wv, wa, b1, w2, b2):
    return pl.pallas_call(
        _node_last_body,
        grid=(NPAD // BN,),
        in_specs=[
            pl.BlockSpec((BN, LATENT), lambda i: (i, 0)),
            pl.BlockSpec((2, BN, 128), lambda i: (0, i, 0)),
            _wspec((LATENT, LATENT)), _wspec((LATENT, LATENT)),
            _wspec((1, LATENT)),
            _wspec((LATENT, LATENT)), _wspec((1, LATENT)),
        ],
        out_specs=pl.BlockSpec((BN, LATENT), lambda i: (i, 0)),
        out_shape=jax.ShapeDtypeStruct((NPAD, LATENT), _f32),
    )(v, parts, wv, wa, b1, w2, b2)


def _dec_call(v, pos18, w1, b1, w2, b2):
    return pl.pallas_call(
        _dec_body,
        grid=(NPAD // BN,),
        in_specs=[
            pl.BlockSpec((BN, LATENT), lambda i: (i, 0)),
            pl.BlockSpec((BN, 18), lambda i: (i, 0)),
            _wspec((LATENT, LATENT)), _wspec((1, LATENT)),
            _wspec((LATENT, 8)), _wspec((1, 8)),
        ],
        out_specs=pl.BlockSpec((BN, 8), lambda i: (i, 0)),
        out_shape=jax.ShapeDtypeStruct((NPAD, 8), _f32),
    )(v, pos18, w1, b1, w2, b2)


def kernel(position_sequence, params, particle_types, senders, receivers,
           n_particles_per_example):
    pos18 = jnp.pad(position_sequence.reshape(N, 18), ((0, NPAD - N), (0, 0)))
    typ = jnp.pad(particle_types.astype(jnp.int32), (0, NPAD - N))
    typ = typ.reshape(NPAD, 1)

    npad_e = EPAD - E
    pad_spread = (jnp.arange(npad_e, dtype=jnp.int32) * 37) % N
    s_pad = jnp.concatenate([senders.astype(jnp.int32), pad_spread])
    r_pad = jnp.concatenate([receivers.astype(jnp.int32), pad_spread])
    idx_gather = jnp.concatenate([s_pad, r_pad]).reshape(-1, CHK)
    pad_sink = N + (jnp.arange(npad_e, dtype=jnp.int32) % (NPAD - N))
    ridx = jnp.concatenate(
        [receivers.astype(jnp.int32), pad_sink]).reshape(-1, CHK)
    zeros_n = jnp.zeros((NPAD, 128), _f32)

    def b2d(b):
        return b.reshape(1, -1)

    (wn1, bn1), (wn2, bn2) = params['enc_node']
    (we1, be1), (we2, be2) = params['enc_edge']
    (wd1, bd1), (wd2, bd2) = params['dec']
    wd2p = jnp.pad(wd2, ((0, 0), (0, 8 - DIMS)))
    bd2p = jnp.pad(bd2, (0, 8 - DIMS))

    ew1 = [sp['edge'][0][0] for sp in params['proc']]
    ws_all = [w[LATENT:2 * LATENT] for w in ew1]
    wr_all = [w[2 * LATENT:] for w in ew1]

    v, p, pos8 = _enc_node_call(pos18, typ, params['type_emb'],
                                wn1, b2d(bn1), wn2, b2d(bn2),
                                ws_all[0], wr_all[0])

    gpos = _sc_gather(pos8, idx_gather)
    e = _enc_edge_call(gpos, we1, b2d(be1), we2, b2d(be2))

    for step in range(MP_STEPS):
        sp = params['proc'][step]
        (w1, b1), (w2, b2) = sp['edge']
        g = _sc_gather(p, idx_gather)
        e_new, e = _edge_step_call(e, g, w1[:LATENT], b2d(b1), w2, b2d(b2))
        parts = _sc_scatter(e_new, ridx, zeros_n)
        (nw1, nb1), (nw2, nb2) = sp['node']
        wv, wa = nw1[:LATENT], nw1[LATENT:]
        if step < MP_STEPS - 1:
            v, p = _node_step_call(v, parts, wv, wa, b2d(nb1), nw2, b2d(nb2),
                                   ws_all[step + 1], wr_all[step + 1])
        else:
            v = _node_last_call(v, parts, wv, wa, b2d(nb1), nw2, b2d(nb2))

    out = _dec_call(v, pos18, wd1, b2d(bd1), wd2p, b2d(bd2p))
    return out[:N, :DIMS]


# speedup vs baseline: 3.1651x; 1.0511x over previous
import functools

import jax
import jax.numpy as jnp
from jax import lax
from jax.experimental import pallas as pl
from jax.experimental.pallas import tpu as pltpu
from jax.experimental.pallas import tpu_sc as plsc

N = 10000
E = 160000
DIMS = 3
LATENT = 64
MP_STEPS = 5
RADIUS = 0.015
NUM_TYPES = 9
TYPE_EMB = 16

NPAD = 10240
EPAD = 163840
NC = 2
NS = 16
NW = NC * NS
CHK = 128
GCH = (2 * EPAD) // (NW * CHK)
SCH = EPAD // (NW * CHK)
NSUB = NPAD // NS

BN = 2048
BE = 2048

_f32 = jnp.float32


def _dot(x, w):
    return jax.lax.dot_general(x, w, (((1,), (0,)), ((), ())),
                               preferred_element_type=_f32)


def _ln(x):
    m = jnp.mean(x, axis=-1, keepdims=True)
    xc = x - m
    v = jnp.mean(xc * xc, axis=-1, keepdims=True)
    return xc * lax.rsqrt(v + 1e-6)


NB = 4
NBS = 2


def _sc_gather(table, idx2d):
    ncols = 128
    nrows = idx2d.shape[0] * CHK
    nch = nrows // (NW * CHK)
    ngr = nch // NB

    @functools.partial(
        pl.kernel,
        out_type=jax.ShapeDtypeStruct((nrows, ncols), _f32),
        mesh=plsc.VectorSubcoreMesh(core_axis_name="c", subcore_axis_name="s",
                                    num_cores=NC, num_subcores=NS),
        scratch_types=[
            pltpu.VMEM((NB, CHK), jnp.int32),
            pltpu.VMEM((NB * CHK, ncols), _f32),
            [pltpu.SemaphoreType.DMA] * NB,
            [pltpu.SemaphoreType.DMA] * NB,
            pltpu.SemaphoreType.DMA,
        ],
    )
    def k(table_hbm, idx_hbm, out_hbm, idx_v, rows_v, semi, semg, semo):
        w = lax.axis_index("s") * NC + lax.axis_index("c")
        base = w * nch

        for b in range(NB):
            pltpu.async_copy(idx_hbm.at[base + b], idx_v.at[b], semi[b])

        @pl.loop(0, ngr)
        def _(g):
            gbase = base + g * NB

            @pl.when(g > 0)
            def _():
                pltpu.make_async_copy(
                    rows_v, out_hbm.at[pl.ds(gbase * CHK, NB * CHK)],
                    semo).wait()

            for b in range(NB):
                pltpu.make_async_copy(idx_hbm.at[gbase + b], idx_v.at[b],
                                      semi[b]).wait()
                pltpu.async_copy(table_hbm.at[idx_v.at[b]],
                                 rows_v.at[pl.ds(b * CHK, CHK)], semg[b])
            for b in range(NB):
                pltpu.make_async_copy(table_hbm.at[idx_v.at[b]],
                                      rows_v.at[pl.ds(b * CHK, CHK)],
                                      semg[b]).wait()

                @pl.when(g + 1 < ngr)
                def _():
                    pltpu.async_copy(idx_hbm.at[gbase + NB + b], idx_v.at[b],
                                     semi[b])

            pltpu.async_copy(rows_v, out_hbm.at[pl.ds(gbase * CHK, NB * CHK)],
                             semo)

        pltpu.make_async_copy(
            rows_v, out_hbm.at[pl.ds((base + nch - NB) * CHK, NB * CHK)],
            semo).wait()

    return k(table, idx2d)


def _sc_scatter(vals, idx2d, zeros_n):
    @functools.partial(
        pl.kernel,
        out_type=jax.ShapeDtypeStruct((NC, NPAD, 128), _f32),
        mesh=plsc.VectorSubcoreMesh(core_axis_name="c", subcore_axis_name="s",
                                    num_cores=NC, num_subcores=NS),
        scratch_types=[
            pltpu.VMEM_SHARED((NPAD, 128), _f32),
            pltpu.VMEM((SCH, CHK), jnp.int32),
            pltpu.VMEM((NBS, CHK, 128), _f32),
            [pltpu.SemaphoreType.DMA] * NBS,
            [pltpu.SemaphoreType.DMA] * NBS,
        ],
    )
    def k(vals_hbm, idx_hbm, zero_hbm, out_hbm, acc_sh, idx_v, vals_v,
          seml, sems):
        c = lax.axis_index("c")
        s = lax.axis_index("s")
        w = c * NS + s
        pltpu.sync_copy(zero_hbm.at[pl.ds(s * NSUB, NSUB)],
                        acc_sh.at[pl.ds(s * NSUB, NSUB)])
        pltpu.sync_copy(idx_hbm.at[pl.ds(w * SCH, SCH)], idx_v)
        for b in range(NBS):
            pltpu.async_copy(
                vals_hbm.at[pl.ds((w * SCH + b) * CHK, CHK)],
                vals_v.at[b], seml[b])
        plsc.subcore_barrier()

        @pl.loop(0, SCH // NBS)
        def _(g):
            jbase = w * SCH + g * NBS
            for b in range(NBS):
                pltpu.make_async_copy(
                    vals_hbm.at[pl.ds((jbase + b) * CHK, CHK)],
                    vals_v.at[b], seml[b]).wait()
                pltpu.async_copy(vals_v.at[b],
                                 acc_sh.at[idx_v.at[g * NBS + b]],
                                 sems[b], add=True)
            for b in range(NBS):
                pltpu.make_async_copy(vals_v.at[b],
                                      acc_sh.at[idx_v.at[g * NBS + b]],
                                      sems[b]).wait()

                @pl.when(g + 1 < SCH // NBS)
                def _():
                    pltpu.async_copy(
                        vals_hbm.at[pl.ds((jbase + NBS + b) * CHK, CHK)],
                        vals_v.at[b], seml[b])

        plsc.subcore_barrier()
        pltpu.sync_copy(acc_sh.at[pl.ds(s * NSUB, NSUB)],
                        out_hbm.at[c].at[pl.ds(s * NSUB, NSUB)])

    return k(vals, idx2d, zeros_n)


def _enc_node_body(pos_ref, typ_ref, emb_ref, w1_ref, b1_ref, w2_ref, b2_ref,
                   v_ref, tab0_ref):
    pos = pos_ref[...]
    nvel = pos[:, 3:18] - pos[:, 0:15]
    mr = pos[:, 15:18]
    dist = jnp.concatenate([mr - 0.1, 0.9 - mr], axis=1)
    distc = jnp.clip(dist * (1.0 / RADIUS), -1.0, 1.0)
    typ = typ_ref[...]
    oh = (typ == lax.broadcasted_iota(jnp.int32, (typ.shape[0], NUM_TYPES), 1))
    te = _dot(oh.astype(_f32), emb_ref[...])
    nf = jnp.concatenate([nvel, distc, te], axis=1)
    h = jnp.maximum(_dot(nf, w1_ref[...]) + b1_ref[...], 0.0)
    v = _ln(_dot(h, w2_ref[...]) + b2_ref[...])
    v_ref[...] = v
    tab0_ref[...] = jnp.concatenate(
        [v, mr, jnp.zeros((mr.shape[0], 61), _f32)], axis=1)


def _edge0_body(gs_ref, gr_ref, we1_ref, be1_ref, we2_ref, be2_ref,
                w1e_ref, w1s_ref, w1r_ref, b1_ref, w2_ref, b2_ref,
                enew_ref, eout_ref):
    vs = gs_ref[:, 0:LATENT]
    vr = gr_ref[:, 0:LATENT]
    rel = (gs_ref[:, LATENT:LATENT + 3]
           - gr_ref[:, LATENT:LATENT + 3]) * (1.0 / RADIUS)
    nrm = jnp.sqrt(jnp.sum(rel * rel, axis=1, keepdims=True))
    ef = jnp.concatenate([rel, nrm], axis=1)
    he = jnp.maximum(_dot(ef, we1_ref[...]) + be1_ref[...], 0.0)
    e = _ln(_dot(he, we2_ref[...]) + be2_ref[...])
    h = jnp.maximum(_dot(e, w1e_ref[...]) + _dot(vs, w1s_ref[...])
                    + _dot(vr, w1r_ref[...]) + b1_ref[...], 0.0)
    en = _ln(_dot(h, w2_ref[...]) + b2_ref[...])
    enew_ref[...] = jnp.concatenate(
        [en, jnp.zeros((en.shape[0], 128 - LATENT), _f32)], axis=1)
    eout_ref[...] = e + en


def _edge_step_body(e_ref, gs_ref, gr_ref, w1e_ref, b1_ref, w2_ref, b2_ref,
                    enew_ref, eout_ref):
    e = e_ref[...]
    h = jnp.maximum(_dot(e, w1e_ref[...]) + gs_ref[:, 0:LATENT]
                    + gr_ref[:, LATENT:] + b1_ref[...], 0.0)
    en = _ln(_dot(h, w2_ref[...]) + b2_ref[...])
    enew_ref[...] = jnp.concatenate(
        [en, jnp.zeros((en.shape[0], 128 - LATENT), _f32)], axis=1)
    eout_ref[...] = e + en


def _node_step_body(v_ref, parts_ref, wv_ref, wa_ref, b1_ref, w2_ref, b2_ref,
                    ws_ref, wr_ref, vout_ref, p_ref):
    v = v_ref[...]
    agg = parts_ref[0, :, 0:LATENT] + parts_ref[1, :, 0:LATENT]
    t = jnp.maximum(_dot(v, wv_ref[...]) + _dot(agg, wa_ref[...])
                    + b1_ref[...], 0.0)
    vo = v + _ln(_dot(t, w2_ref[...]) + b2_ref[...])
    vout_ref[...] = vo
    p_ref[...] = jnp.concatenate(
        [_dot(vo, ws_ref[...]), _dot(vo, wr_ref[...])], axis=1)


def _node_last_body(v_ref, parts_ref, wv_ref, wa_ref, b1_ref, w2_ref, b2_ref,
                    vout_ref):
    v = v_ref[...]
    agg = parts_ref[0, :, 0:LATENT] + parts_ref[1, :, 0:LATENT]
    t = jnp.maximum(_dot(v, wv_ref[...]) + _dot(agg, wa_ref[...])
                    + b1_ref[...], 0.0)
    vout_ref[...] = v + _ln(_dot(t, w2_ref[...]) + b2_ref[...])


def _dec_body(v_ref, pos_ref, w1_ref, b1_ref, w2_ref, b2_ref, out_ref):
    v = v_ref[...]
    t = jnp.maximum(_dot(v, w1_ref[...]) + b1_ref[...], 0.0)
    acc = _dot(t, w2_ref[...]) + b2_ref[...]
    pos = pos_ref[...]
    mr = pos[:, 15:18]
    pv = pos[:, 12:15]
    out3 = mr + (mr - pv) + acc[:, 0:3]
    out_ref[...] = jnp.concatenate([out3, acc[:, 3:8]], axis=1)


def _wspec(shape):
    return pl.BlockSpec(shape, lambda i: tuple(0 for _ in shape))


def _enc_node_call(pos18, typ, emb, w1, b1, w2, b2):
    return pl.pallas_call(
        _enc_node_body,
        grid=(NPAD // BN,),
        in_specs=[
            pl.BlockSpec((BN, 18), lambda i: (i, 0)),
            pl.BlockSpec((BN, 1), lambda i: (i, 0)),
            _wspec((NUM_TYPES, TYPE_EMB)),
            _wspec((37, LATENT)), _wspec((1, LATENT)),
            _wspec((LATENT, LATENT)), _wspec((1, LATENT)),
        ],
        out_specs=[
            pl.BlockSpec((BN, LATENT), lambda i: (i, 0)),
            pl.BlockSpec((BN, 128), lambda i: (i, 0)),
        ],
        out_shape=[
            jax.ShapeDtypeStruct((NPAD, LATENT), _f32),
            jax.ShapeDtypeStruct((NPAD, 128), _f32),
        ],
    )(pos18, typ, emb, w1, b1, w2, b2)


def _edge0_call(g, we1, be1, we2, be2, w1e, w1s, w1r, b1, w2, b2):
    return pl.pallas_call(
        _edge0_body,
        grid=(EPAD // BE,),
        in_specs=[
            pl.BlockSpec((BE, 128), lambda i: (i, 0)),
            pl.BlockSpec((BE, 128), lambda i: (i + EPAD // BE, 0)),
            _wspec((4, LATENT)), _wspec((1, LATENT)),
            _wspec((LATENT, LATENT)), _wspec((1, LATENT)),
            _wspec((LATENT, LATENT)), _wspec((LATENT, LATENT)),
            _wspec((LATENT, LATENT)), _wspec((1, LATENT)),
            _wspec((LATENT, LATENT)), _wspec((1, LATENT)),
        ],
        out_specs=[
            pl.BlockSpec((BE, 128), lambda i: (i, 0)),
            pl.BlockSpec((BE, LATENT), lambda i: (i, 0)),
        ],
        out_shape=[
            jax.ShapeDtypeStruct((EPAD, 128), _f32),
            jax.ShapeDtypeStruct((EPAD, LATENT), _f32),
        ],
    )(g, g, we1, be1, we2, be2, w1e, w1s, w1r, b1, w2, b2)


def _edge_step_call(e, g, w1e, b1, w2, b2):
    return pl.pallas_call(
        _edge_step_body,
        grid=(EPAD // BE,),
        in_specs=[
            pl.BlockSpec((BE, LATENT), lambda i: (i, 0)),
            pl.BlockSpec((BE, 128), lambda i: (i, 0)),
            pl.BlockSpec((BE, 128), lambda i: (i + EPAD // BE, 0)),
            _wspec((LATENT, LATENT)), _wspec((1, LATENT)),
            _wspec((LATENT, LATENT)), _wspec((1, LATENT)),
        ],
        out_specs=[
            pl.BlockSpec((BE, 128), lambda i: (i, 0)),
            pl.BlockSpec((BE, LATENT), lambda i: (i, 0)),
        ],
        out_shape=[
            jax.ShapeDtypeStruct((EPAD, 128), _f32),
            jax.ShapeDtypeStruct((EPAD, LATENT), _f32),
        ],
    )(e, g, g, w1e, b1, w2, b2)


def _node_step_call(v, parts, wv, wa, b1, w2, b2, ws, wr):
    return pl.pallas_call(
        _node_step_body,
        grid=(NPAD // BN,),
        in_specs=[
            pl.BlockSpec((BN, LATENT), lambda i: (i, 0)),
            pl.BlockSpec((2, BN, 128), lambda i: (0, i, 0)),
            _wspec((LATENT, LATENT)), _wspec((LATENT, LATENT)),
            _wspec((1, LATENT)),
            _wspec((LATENT, LATENT)), _wspec((1, LATENT)),
            _wspec((LATENT, LATENT)), _wspec((LATENT, LATENT)),
        ],
        out_specs=[
            pl.BlockSpec((BN, LATENT), lambda i: (i, 0)),
            pl.BlockSpec((BN, 128), lambda i: (i, 0)),
        ],
        out_shape=[
            jax.ShapeDtypeStruct((NPAD, LATENT), _f32),
            jax.ShapeDtypeStruct((NPAD, 128), _f32),
        ],
    )(v, parts, wv, wa, b1, w2, b2, ws, wr)


def _node_last_call(v, parts, wv, wa, b1, w2, b2):
    return pl.pallas_call(
        _node_last_body,
        grid=(NPAD // BN,),
        in_specs=[
            pl.BlockSpec((BN, LATENT), lambda i: (i, 0)),
            pl.BlockSpec((2, BN, 128), lambda i: (0, i, 0)),
            _wspec((LATENT, LATENT)), _wspec((LATENT, LATENT)),
            _wspec((1, LATENT)),
            _wspec((LATENT, LATENT)), _wspec((1, LATENT)),
        ],
        out_specs=pl.BlockSpec((BN, LATENT), lambda i: (i, 0)),
        out_shape=jax.ShapeDtypeStruct((NPAD, LATENT), _f32),
    )(v, parts, wv, wa, b1, w2, b2)


def _dec_call(v, pos18, w1, b1, w2, b2):
    return pl.pallas_call(
        _dec_body,
        grid=(NPAD // BN,),
        in_specs=[
            pl.BlockSpec((BN, LATENT), lambda i: (i, 0)),
            pl.BlockSpec((BN, 18), lambda i: (i, 0)),
            _wspec((LATENT, LATENT)), _wspec((1, LATENT)),
            _wspec((LATENT, 8)), _wspec((1, 8)),
        ],
        out_specs=pl.BlockSpec((BN, 8), lambda i: (i, 0)),
        out_shape=jax.ShapeDtypeStruct((NPAD, 8), _f32),
    )(v, pos18, w1, b1, w2, b2)


def kernel(position_sequence, params, particle_types, senders, receivers,
           n_particles_per_example):
    pos18 = jnp.pad(position_sequence.reshape(N, 18), ((0, NPAD - N), (0, 0)))
    typ = jnp.pad(particle_types.astype(jnp.int32), (0, NPAD - N))
    typ = typ.reshape(NPAD, 1)

    npad_e = EPAD - E
    pad_spread = (jnp.arange(npad_e, dtype=jnp.int32) * 37) % N
    s_pad = jnp.concatenate([senders.astype(jnp.int32), pad_spread])
    r_pad = jnp.concatenate([receivers.astype(jnp.int32), pad_spread])
    idx_gather = jnp.concatenate([s_pad, r_pad]).reshape(-1, CHK)
    pad_sink = N + (jnp.arange(npad_e, dtype=jnp.int32) % (NPAD - N))
    ridx = jnp.concatenate(
        [receivers.astype(jnp.int32), pad_sink]).reshape(-1, CHK)
    zeros_n = jnp.zeros((NPAD, 128), _f32)

    def b2d(b):
        return b.reshape(1, -1)

    (wn1, bn1), (wn2, bn2) = params['enc_node']
    (we1, be1), (we2, be2) = params['enc_edge']
    (wd1, bd1), (wd2, bd2) = params['dec']
    wd2p = jnp.pad(wd2, ((0, 0), (0, 8 - DIMS)))
    bd2p = jnp.pad(bd2, (0, 8 - DIMS))

    ew1 = [sp['edge'][0][0] for sp in params['proc']]
    ws_all = [w[LATENT:2 * LATENT] for w in ew1]
    wr_all = [w[2 * LATENT:] for w in ew1]

    v, tab0 = _enc_node_call(pos18, typ, params['type_emb'],
                             wn1, b2d(bn1), wn2, b2d(bn2))

    p = tab0
    for step in range(MP_STEPS):
        sp = params['proc'][step]
        (w1, b1), (w2, b2) = sp['edge']
        g = _sc_gather(p, idx_gather)
        if step == 0:
            e_new, e = _edge0_call(g, we1, b2d(be1), we2, b2d(be2),
                                   w1[:LATENT], ws_all[0], wr_all[0],
                                   b2d(b1), w2, b2d(b2))
        else:
            e_new, e = _edge_step_call(e, g, w1[:LATENT], b2d(b1), w2,
                                       b2d(b2))
        parts = _sc_scatter(e_new, ridx, zeros_n)
        (nw1, nb1), (nw2, nb2) = sp['node']
        wv, wa = nw1[:LATENT], nw1[LATENT:]
        if step < MP_STEPS - 1:
            v, p = _node_step_call(v, parts, wv, wa, b2d(nb1), nw2, b2d(nb2),
                                   ws_all[step + 1], wr_all[step + 1])
        else:
            v = _node_last_call(v, parts, wv, wa, b2d(nb1), nw2, b2d(nb2))

    out = _dec_call(v, pos18, wd1, b2d(bd1), wd2p, b2d(bd2p))
    return out[:N, :DIMS]


# speedup vs baseline: 3.4361x; 1.0856x over previous
import functools

import jax
import jax.numpy as jnp
from jax import lax
from jax.experimental import pallas as pl
from jax.experimental.pallas import tpu as pltpu
from jax.experimental.pallas import tpu_sc as plsc

N = 10000
E = 160000
DIMS = 3
LATENT = 64
MP_STEPS = 5
RADIUS = 0.015
NUM_TYPES = 9
TYPE_EMB = 16

NPAD = 10240
EPAD = 163840
NC = 2
NS = 16
NW = NC * NS
CHK = 128
GCH = (2 * EPAD) // (NW * CHK)
SCH = EPAD // (NW * CHK)
NSUB = NPAD // NS

BN = 2048
BE = 2048
EH = EPAD // 2

_f32 = jnp.float32


def _dot(x, w):
    return jax.lax.dot_general(x, w, (((1,), (0,)), ((), ())),
                               preferred_element_type=_f32)


def _ln(x):
    m = jnp.mean(x, axis=-1, keepdims=True)
    xc = x - m
    v = jnp.mean(xc * xc, axis=-1, keepdims=True)
    return xc * lax.rsqrt(v + 1e-6)


NB = 4
NBS = 2


def _sc_gather(table, idx2d):
    ncols = 128
    nrows = idx2d.shape[0] * CHK
    nch = nrows // (NW * CHK)
    ngr = nch // NB

    @functools.partial(
        pl.kernel,
        out_type=jax.ShapeDtypeStruct((nrows, ncols), _f32),
        mesh=plsc.VectorSubcoreMesh(core_axis_name="c", subcore_axis_name="s",
                                    num_cores=NC, num_subcores=NS),
        scratch_types=[
            pltpu.VMEM((NB, CHK), jnp.int32),
            pltpu.VMEM((NB * CHK, ncols), _f32),
            [pltpu.SemaphoreType.DMA] * NB,
            [pltpu.SemaphoreType.DMA] * NB,
            pltpu.SemaphoreType.DMA,
        ],
    )
    def k(table_hbm, idx_hbm, out_hbm, idx_v, rows_v, semi, semg, semo):
        w = lax.axis_index("s") * NC + lax.axis_index("c")
        base = w * nch

        for b in range(NB):
            pltpu.async_copy(idx_hbm.at[base + b], idx_v.at[b], semi[b])

        @pl.loop(0, ngr)
        def _(g):
            gbase = base + g * NB

            @pl.when(g > 0)
            def _():
                pltpu.make_async_copy(
                    rows_v, out_hbm.at[pl.ds(gbase * CHK, NB * CHK)],
                    semo).wait()

            for b in range(NB):
                pltpu.make_async_copy(idx_hbm.at[gbase + b], idx_v.at[b],
                                      semi[b]).wait()
                pltpu.async_copy(table_hbm.at[idx_v.at[b]],
                                 rows_v.at[pl.ds(b * CHK, CHK)], semg[b])
            for b in range(NB):
                pltpu.make_async_copy(table_hbm.at[idx_v.at[b]],
                                      rows_v.at[pl.ds(b * CHK, CHK)],
                                      semg[b]).wait()

                @pl.when(g + 1 < ngr)
                def _():
                    pltpu.async_copy(idx_hbm.at[gbase + NB + b], idx_v.at[b],
                                     semi[b])

            pltpu.async_copy(rows_v, out_hbm.at[pl.ds(gbase * CHK, NB * CHK)],
                             semo)

        pltpu.make_async_copy(
            rows_v, out_hbm.at[pl.ds((base + nch - NB) * CHK, NB * CHK)],
            semo).wait()

    return k(table, idx2d)


def _sc_scatter(vals, idx2d, zeros_n):
    sch = vals.shape[0] // (NW * CHK)
    idx3d = idx2d.reshape(NW, sch, CHK)

    @functools.partial(
        pl.kernel,
        out_type=jax.ShapeDtypeStruct((NC, NPAD, 128), _f32),
        mesh=plsc.VectorSubcoreMesh(core_axis_name="c", subcore_axis_name="s",
                                    num_cores=NC, num_subcores=NS),
        scratch_types=[
            pltpu.VMEM_SHARED((NPAD, 128), _f32),
            pltpu.VMEM((sch, CHK), jnp.int32),
            pltpu.VMEM((NBS, CHK, 128), _f32),
            [pltpu.SemaphoreType.DMA] * NBS,
            [pltpu.SemaphoreType.DMA] * NBS,
        ],
    )
    def k(vals_hbm, idx_hbm, zero_hbm, out_hbm, acc_sh, idx_v, vals_v,
          seml, sems):
        c = lax.axis_index("c")
        s = lax.axis_index("s")
        w = c * NS + s
        pltpu.sync_copy(zero_hbm.at[pl.ds(s * NSUB, NSUB)],
                        acc_sh.at[pl.ds(s * NSUB, NSUB)])
        pltpu.sync_copy(idx_hbm.at[w], idx_v)
        for b in range(NBS):
            pltpu.async_copy(
                vals_hbm.at[pl.ds((w * sch + b) * CHK, CHK)],
                vals_v.at[b], seml[b])
        plsc.subcore_barrier()

        @pl.loop(0, sch // NBS)
        def _(g):
            jbase = w * sch + g * NBS
            for b in range(NBS):
                pltpu.make_async_copy(
                    vals_hbm.at[pl.ds((jbase + b) * CHK, CHK)],
                    vals_v.at[b], seml[b]).wait()
                pltpu.async_copy(vals_v.at[b],
                                 acc_sh.at[idx_v.at[g * NBS + b]],
                                 sems[b], add=True)
            for b in range(NBS):
                pltpu.make_async_copy(vals_v.at[b],
                                      acc_sh.at[idx_v.at[g * NBS + b]],
                                      sems[b]).wait()

                @pl.when(g + 1 < sch // NBS)
                def _():
                    pltpu.async_copy(
                        vals_hbm.at[pl.ds((jbase + NBS + b) * CHK, CHK)],
                        vals_v.at[b], seml[b])

        plsc.subcore_barrier()
        pltpu.sync_copy(acc_sh.at[pl.ds(s * NSUB, NSUB)],
                        out_hbm.at[c].at[pl.ds(s * NSUB, NSUB)])

    return k(vals, idx3d, zeros_n)


def _enc_node_body(pos_ref, typ_ref, emb_ref, w1_ref, b1_ref, w2_ref, b2_ref,
                   v_ref, tab0_ref):
    pos = pos_ref[...]
    nvel = pos[:, 3:18] - pos[:, 0:15]
    mr = pos[:, 15:18]
    dist = jnp.concatenate([mr - 0.1, 0.9 - mr], axis=1)
    distc = jnp.clip(dist * (1.0 / RADIUS), -1.0, 1.0)
    typ = typ_ref[...]
    oh = (typ == lax.broadcasted_iota(jnp.int32, (typ.shape[0], NUM_TYPES), 1))
    te = _dot(oh.astype(_f32), emb_ref[...])
    nf = jnp.concatenate([nvel, distc, te], axis=1)
    h = jnp.maximum(_dot(nf, w1_ref[...]) + b1_ref[...], 0.0)
    v = _ln(_dot(h, w2_ref[...]) + b2_ref[...])
    v_ref[...] = v
    tab0_ref[...] = jnp.concatenate(
        [v, mr, jnp.zeros((mr.shape[0], 61), _f32)], axis=1)


def _edge0_body(gs_ref, gr_ref, we1_ref, be1_ref, we2_ref, be2_ref,
                w1e_ref, w1s_ref, w1r_ref, b1_ref, w2_ref, b2_ref,
                enew_ref, eout_ref):
    vs = gs_ref[:, 0:LATENT]
    vr = gr_ref[:, 0:LATENT]
    rel = (gs_ref[:, LATENT:LATENT + 3]
           - gr_ref[:, LATENT:LATENT + 3]) * (1.0 / RADIUS)
    nrm = jnp.sqrt(jnp.sum(rel * rel, axis=1, keepdims=True))
    ef = jnp.concatenate([rel, nrm], axis=1)
    he = jnp.maximum(_dot(ef, we1_ref[...]) + be1_ref[...], 0.0)
    e = _ln(_dot(he, we2_ref[...]) + be2_ref[...])
    h = jnp.maximum(_dot(e, w1e_ref[...]) + _dot(vs, w1s_ref[...])
                    + _dot(vr, w1r_ref[...]) + b1_ref[...], 0.0)
    en = _ln(_dot(h, w2_ref[...]) + b2_ref[...])
    enew_ref[...] = jnp.concatenate(
        [en, jnp.zeros((en.shape[0], 128 - LATENT), _f32)], axis=1)
    eout_ref[...] = e + en


def _edge_step_body(e_ref, gs_ref, gr_ref, w1e_ref, b1_ref, w2_ref, b2_ref,
                    enew_ref, eout_ref):
    e = e_ref[...]
    h = jnp.maximum(_dot(e, w1e_ref[...]) + gs_ref[:, 0:LATENT]
                    + gr_ref[:, LATENT:] + b1_ref[...], 0.0)
    en = _ln(_dot(h, w2_ref[...]) + b2_ref[...])
    enew_ref[...] = jnp.concatenate(
        [en, jnp.zeros((en.shape[0], 128 - LATENT), _f32)], axis=1)
    eout_ref[...] = e + en


def _node_step_body(v_ref, parts_ref, partsb_ref, wv_ref, wa_ref, b1_ref,
                    w2_ref, b2_ref, ws_ref, wr_ref, vout_ref, p_ref):
    v = v_ref[...]
    agg = (parts_ref[0, :, 0:LATENT] + parts_ref[1, :, 0:LATENT]
           + partsb_ref[0, :, 0:LATENT] + partsb_ref[1, :, 0:LATENT])
    t = jnp.maximum(_dot(v, wv_ref[...]) + _dot(agg, wa_ref[...])
                    + b1_ref[...], 0.0)
    vo = v + _ln(_dot(t, w2_ref[...]) + b2_ref[...])
    vout_ref[...] = vo
    p_ref[...] = jnp.concatenate(
        [_dot(vo, ws_ref[...]), _dot(vo, wr_ref[...])], axis=1)


def _node_last_body(v_ref, parts_ref, partsb_ref, wv_ref, wa_ref, b1_ref,
                    w2_ref, b2_ref, vout_ref):
    v = v_ref[...]
    agg = (parts_ref[0, :, 0:LATENT] + parts_ref[1, :, 0:LATENT]
           + partsb_ref[0, :, 0:LATENT] + partsb_ref[1, :, 0:LATENT])
    t = jnp.maximum(_dot(v, wv_ref[...]) + _dot(agg, wa_ref[...])
                    + b1_ref[...], 0.0)
    vout_ref[...] = v + _ln(_dot(t, w2_ref[...]) + b2_ref[...])


def _dec_body(v_ref, pos_ref, w1_ref, b1_ref, w2_ref, b2_ref, out_ref):
    v = v_ref[...]
    t = jnp.maximum(_dot(v, w1_ref[...]) + b1_ref[...], 0.0)
    acc = _dot(t, w2_ref[...]) + b2_ref[...]
    pos = pos_ref[...]
    mr = pos[:, 15:18]
    pv = pos[:, 12:15]
    out3 = mr + (mr - pv) + acc[:, 0:3]
    out_ref[...] = jnp.concatenate([out3, acc[:, 3:8]], axis=1)


def _wspec(shape):
    return pl.BlockSpec(shape, lambda i: tuple(0 for _ in shape))


def _enc_node_call(pos18, typ, emb, w1, b1, w2, b2):
    return pl.pallas_call(
        _enc_node_body,
        grid=(NPAD // BN,),
        in_specs=[
            pl.BlockSpec((BN, 18), lambda i: (i, 0)),
            pl.BlockSpec((BN, 1), lambda i: (i, 0)),
            _wspec((NUM_TYPES, TYPE_EMB)),
            _wspec((37, LATENT)), _wspec((1, LATENT)),
            _wspec((LATENT, LATENT)), _wspec((1, LATENT)),
        ],
        out_specs=[
            pl.BlockSpec((BN, LATENT), lambda i: (i, 0)),
            pl.BlockSpec((BN, 128), lambda i: (i, 0)),
        ],
        out_shape=[
            jax.ShapeDtypeStruct((NPAD, LATENT), _f32),
            jax.ShapeDtypeStruct((NPAD, 128), _f32),
        ],
    )(pos18, typ, emb, w1, b1, w2, b2)


def _edge0_call(g, we1, be1, we2, be2, w1e, w1s, w1r, b1, w2, b2):
    return pl.pallas_call(
        _edge0_body,
        grid=(EH // BE,),
        in_specs=[
            pl.BlockSpec((BE, 128), lambda i: (i, 0)),
            pl.BlockSpec((BE, 128), lambda i: (i + EH // BE, 0)),
            _wspec((4, LATENT)), _wspec((1, LATENT)),
            _wspec((LATENT, LATENT)), _wspec((1, LATENT)),
            _wspec((LATENT, LATENT)), _wspec((LATENT, LATENT)),
            _wspec((LATENT, LATENT)), _wspec((1, LATENT)),
            _wspec((LATENT, LATENT)), _wspec((1, LATENT)),
        ],
        out_specs=[
            pl.BlockSpec((BE, 128), lambda i: (i, 0)),
            pl.BlockSpec((BE, LATENT), lambda i: (i, 0)),
        ],
        out_shape=[
            jax.ShapeDtypeStruct((EH, 128), _f32),
            jax.ShapeDtypeStruct((EH, LATENT), _f32),
        ],
    )(g, g, we1, be1, we2, be2, w1e, w1s, w1r, b1, w2, b2)


def _edge_step_call(e, g, w1e, b1, w2, b2):
    return pl.pallas_call(
        _edge_step_body,
        grid=(EH // BE,),
        in_specs=[
            pl.BlockSpec((BE, LATENT), lambda i: (i, 0)),
            pl.BlockSpec((BE, 128), lambda i: (i, 0)),
            pl.BlockSpec((BE, 128), lambda i: (i + EH // BE, 0)),
            _wspec((LATENT, LATENT)), _wspec((1, LATENT)),
            _wspec((LATENT, LATENT)), _wspec((1, LATENT)),
        ],
        out_specs=[
            pl.BlockSpec((BE, 128), lambda i: (i, 0)),
            pl.BlockSpec((BE, LATENT), lambda i: (i, 0)),
        ],
        out_shape=[
            jax.ShapeDtypeStruct((EH, 128), _f32),
            jax.ShapeDtypeStruct((EH, LATENT), _f32),
        ],
    )(e, g, g, w1e, b1, w2, b2)


def _node_step_call(v, parts, partsb, wv, wa, b1, w2, b2, ws, wr):
    return pl.pallas_call(
        _node_step_body,
        grid=(NPAD // BN,),
        in_specs=[
            pl.BlockSpec((BN, LATENT), lambda i: (i, 0)),
            pl.BlockSpec((2, BN, 128), lambda i: (0, i, 0)),
            pl.BlockSpec((2, BN, 128), lambda i: (0, i, 0)),
            _wspec((LATENT, LATENT)), _wspec((LATENT, LATENT)),
            _wspec((1, LATENT)),
            _wspec((LATENT, LATENT)), _wspec((1, LATENT)),
            _wspec((LATENT, LATENT)), _wspec((LATENT, LATENT)),
        ],
        out_specs=[
            pl.BlockSpec((BN, LATENT), lambda i: (i, 0)),
            pl.BlockSpec((BN, 128), lambda i: (i, 0)),
        ],
        out_shape=[
            jax.ShapeDtypeStruct((NPAD, LATENT), _f32),
            jax.ShapeDtypeStruct((NPAD, 128), _f32),
        ],
    )(v, parts, partsb, wv, wa, b1, w2, b2, ws, wr)


def _node_last_call(v, parts, partsb, wv, wa, b1, w2, b2):
    return pl.pallas_call(
        _node_last_body,
        grid=(NPAD // BN,),
        in_specs=[
            pl.BlockSpec((BN, LATENT), lambda i: (i, 0)),
            pl.BlockSpec((2, BN, 128), lambda i: (0, i, 0)),
            pl.BlockSpec((2, BN, 128), lambda i: (0, i, 0)),
            _wspec((LATENT, LATENT)), _wspec((LATENT, LATENT)),
            _wspec((1, LATENT)),
            _wspec((LATENT, LATENT)), _wspec((1, LATENT)),
        ],
        out_specs=pl.BlockSpec((BN, LATENT), lambda i: (i, 0)),
        out_shape=jax.ShapeDtypeStruct((NPAD, LATENT), _f32),
    )(v, parts, partsb, wv, wa, b1, w2, b2)


def _dec_call(v, pos18, w1, b1, w2, b2):
    return pl.pallas_call(
        _dec_body,
        grid=(NPAD // BN,),
        in_specs=[
            pl.BlockSpec((BN, LATENT), lambda i: (i, 0)),
            pl.BlockSpec((BN, 18), lambda i: (i, 0)),
            _wspec((LATENT, LATENT)), _wspec((1, LATENT)),
            _wspec((LATENT, 8)), _wspec((1, 8)),
        ],
        out_specs=pl.BlockSpec((BN, 8), lambda i: (i, 0)),
        out_shape=jax.ShapeDtypeStruct((NPAD, 8), _f32),
    )(v, pos18, w1, b1, w2, b2)


def kernel(position_sequence, params, particle_types, senders, receivers,
           n_particles_per_example):
    pos18 = jnp.pad(position_sequence.reshape(N, 18), ((0, NPAD - N), (0, 0)))
    typ = jnp.pad(particle_types.astype(jnp.int32), (0, NPAD - N))
    typ = typ.reshape(NPAD, 1)

    npad_e = EPAD - E
    pad_spread = (jnp.arange(npad_e, dtype=jnp.int32) * 37) % N
    s_pad = jnp.concatenate([senders.astype(jnp.int32), pad_spread])
    r_pad = jnp.concatenate([receivers.astype(jnp.int32), pad_spread])
    pad_sink = N + (jnp.arange(npad_e, dtype=jnp.int32) % (NPAD - N))
    r_sink = jnp.concatenate([receivers.astype(jnp.int32), pad_sink])
    idx_a = jnp.concatenate([s_pad[:EH], r_pad[:EH]]).reshape(-1, CHK)
    idx_b = jnp.concatenate([s_pad[EH:], r_pad[EH:]]).reshape(-1, CHK)
    ridx_a = r_sink[:EH].reshape(-1, CHK)
    ridx_b = r_sink[EH:].reshape(-1, CHK)
    zeros_n = jnp.zeros((NPAD, 128), _f32)

    def b2d(b):
        return b.reshape(1, -1)

    (wn1, bn1), (wn2, bn2) = params['enc_node']
    (we1, be1), (we2, be2) = params['enc_edge']
    (wd1, bd1), (wd2, bd2) = params['dec']
    wd2p = jnp.pad(wd2, ((0, 0), (0, 8 - DIMS)))
    bd2p = jnp.pad(bd2, (0, 8 - DIMS))

    ew1 = [sp['edge'][0][0] for sp in params['proc']]
    ws_all = [w[LATENT:2 * LATENT] for w in ew1]
    wr_all = [w[2 * LATENT:] for w in ew1]

    v, tab0 = _enc_node_call(pos18, typ, params['type_emb'],
                             wn1, b2d(bn1), wn2, b2d(bn2))

    p = tab0
    ea = eb = None
    for step in range(MP_STEPS):
        sp = params['proc'][step]
        (w1, b1), (w2, b2) = sp['edge']
        ga = _sc_gather(p, idx_a)
        gb = _sc_gather(p, idx_b)
        if step == 0:
            ena, ea = _edge0_call(ga, we1, b2d(be1), we2, b2d(be2),
                                  w1[:LATENT], ws_all[0], wr_all[0],
                                  b2d(b1), w2, b2d(b2))
            enb, eb = _edge0_call(gb, we1, b2d(be1), we2, b2d(be2),
                                  w1[:LATENT], ws_all[0], wr_all[0],
                                  b2d(b1), w2, b2d(b2))
        else:
            ena, ea = _edge_step_call(ea, ga, w1[:LATENT], b2d(b1), w2,
                                      b2d(b2))
            enb, eb = _edge_step_call(eb, gb, w1[:LATENT], b2d(b1), w2,
                                      b2d(b2))
        pa = _sc_scatter(ena, ridx_a, zeros_n)
        pb = _sc_scatter(enb, ridx_b, zeros_n)
        (nw1, nb1), (nw2, nb2) = sp['node']
        wv, wa = nw1[:LATENT], nw1[LATENT:]
        if step < MP_STEPS - 1:
            v, p = _node_step_call(v, pa, pb, wv, wa, b2d(nb1), nw2,
                                   b2d(nb2), ws_all[step + 1],
                                   wr_all[step + 1])
        else:
            v = _node_last_call(v, pa, pb, wv, wa, b2d(nb1), nw2, b2d(nb2))

    out = _dec_call(v, pos18, wd1, b2d(bd1), wd2p, b2d(bd2p))
    return out[:N, :DIMS]


# speedup vs baseline: 3.7385x; 1.0880x over previous
import functools

import jax
import jax.numpy as jnp
from jax import lax
from jax.experimental import pallas as pl
from jax.experimental.pallas import tpu as pltpu
from jax.experimental.pallas import tpu_sc as plsc

N = 10000
E = 160000
DIMS = 3
LATENT = 64
MP_STEPS = 5
RADIUS = 0.015
NUM_TYPES = 9
TYPE_EMB = 16

NPAD = 10240
EPAD = 163840
NC = 2
NS = 16
NW = NC * NS
CHK = 128
GCH = (2 * EPAD) // (NW * CHK)
SCH = EPAD // (NW * CHK)
NSUB = NPAD // NS

BN = 2048
BE = 2048
EH = EPAD // 2

_f32 = jnp.float32


def _dot(x, w):
    return jax.lax.dot_general(x, w, (((1,), (0,)), ((), ())),
                               preferred_element_type=_f32)


def _ln(x):
    m = jnp.mean(x, axis=-1, keepdims=True)
    xc = x - m
    v = jnp.mean(xc * xc, axis=-1, keepdims=True)
    return xc * lax.rsqrt(v + 1e-6)


NB = 2
NBS = 2


def _sc_gather(table, idx2d):
    ncols = 128
    nrows = idx2d.shape[0] * CHK
    nch = nrows // (NW * CHK)
    ngr = nch // NB

    @functools.partial(
        pl.kernel,
        out_type=jax.ShapeDtypeStruct((nrows, ncols), _f32),
        mesh=plsc.VectorSubcoreMesh(core_axis_name="c", subcore_axis_name="s",
                                    num_cores=NC, num_subcores=NS),
        scratch_types=[
            pltpu.VMEM_SHARED((NPAD, 128), _f32),
            pltpu.VMEM((NB, CHK), jnp.int32),
            pltpu.VMEM((NB * CHK, ncols), _f32),
            [pltpu.SemaphoreType.DMA] * NB,
            [pltpu.SemaphoreType.DMA] * NB,
            pltpu.SemaphoreType.DMA,
        ],
    )
    def k(table_hbm, idx_hbm, out_hbm, tab_sh, idx_v, rows_v, semi, semg,
          semo):
        c = lax.axis_index("c")
        s = lax.axis_index("s")
        w = s * NC + c
        base = w * nch

        pltpu.sync_copy(table_hbm.at[pl.ds(s * NSUB, NSUB)],
                        tab_sh.at[pl.ds(s * NSUB, NSUB)])
        for b in range(NB):
            pltpu.async_copy(idx_hbm.at[base + b], idx_v.at[b], semi[b])
        plsc.subcore_barrier()

        @pl.loop(0, ngr)
        def _(g):
            gbase = base + g * NB

            @pl.when(g > 0)
            def _():
                pltpu.make_async_copy(
                    rows_v, out_hbm.at[pl.ds(gbase * CHK, NB * CHK)],
                    semo).wait()

            for b in range(NB):
                pltpu.make_async_copy(idx_hbm.at[gbase + b], idx_v.at[b],
                                      semi[b]).wait()
                pltpu.async_copy(tab_sh.at[idx_v.at[b]],
                                 rows_v.at[pl.ds(b * CHK, CHK)], semg[b])
            for b in range(NB):
                pltpu.make_async_copy(tab_sh.at[idx_v.at[b]],
                                      rows_v.at[pl.ds(b * CHK, CHK)],
                                      semg[b]).wait()

                @pl.when(g + 1 < ngr)
                def _():
                    pltpu.async_copy(idx_hbm.at[gbase + NB + b], idx_v.at[b],
                                     semi[b])

            pltpu.async_copy(rows_v, out_hbm.at[pl.ds(gbase * CHK, NB * CHK)],
                             semo)

        pltpu.make_async_copy(
            rows_v, out_hbm.at[pl.ds((base + nch - NB) * CHK, NB * CHK)],
            semo).wait()

    return k(table, idx2d)


def _sc_scatter(vals, idx2d, zeros_n):
    sch = vals.shape[0] // (NW * CHK)
    idx3d = idx2d.reshape(NW, sch, CHK)

    @functools.partial(
        pl.kernel,
        out_type=jax.ShapeDtypeStruct((NC, NPAD, 128), _f32),
        mesh=plsc.VectorSubcoreMesh(core_axis_name="c", subcore_axis_name="s",
                                    num_cores=NC, num_subcores=NS),
        scratch_types=[
            pltpu.VMEM_SHARED((NPAD, 128), _f32),
            pltpu.VMEM((sch, CHK), jnp.int32),
            pltpu.VMEM((NBS, CHK, 128), _f32),
            [pltpu.SemaphoreType.DMA] * NBS,
            [pltpu.SemaphoreType.DMA] * NBS,
        ],
    )
    def k(vals_hbm, idx_hbm, zero_hbm, out_hbm, acc_sh, idx_v, vals_v,
          seml, sems):
        c = lax.axis_index("c")
        s = lax.axis_index("s")
        w = c * NS + s
        pltpu.sync_copy(zero_hbm.at[pl.ds(s * NSUB, NSUB)],
                        acc_sh.at[pl.ds(s * NSUB, NSUB)])
        pltpu.sync_copy(idx_hbm.at[w], idx_v)
        for b in range(NBS):
            pltpu.async_copy(
                vals_hbm.at[pl.ds((w * sch + b) * CHK, CHK)],
                vals_v.at[b], seml[b])
        plsc.subcore_barrier()

        @pl.loop(0, sch // NBS)
        def _(g):
            jbase = w * sch + g * NBS
            for b in range(NBS):
                pltpu.make_async_copy(
                    vals_hbm.at[pl.ds((jbase + b) * CHK, CHK)],
                    vals_v.at[b], seml[b]).wait()
                pltpu.async_copy(vals_v.at[b],
                                 acc_sh.at[idx_v.at[g * NBS + b]],
                                 sems[b], add=True)
            for b in range(NBS):
                pltpu.make_async_copy(vals_v.at[b],
                                      acc_sh.at[idx_v.at[g * NBS + b]],
                                      sems[b]).wait()

                @pl.when(g + 1 < sch // NBS)
                def _():
                    pltpu.async_copy(
                        vals_hbm.at[pl.ds((jbase + NBS + b) * CHK, CHK)],
                        vals_v.at[b], seml[b])

        plsc.subcore_barrier()
        pltpu.sync_copy(acc_sh.at[pl.ds(s * NSUB, NSUB)],
                        out_hbm.at[c].at[pl.ds(s * NSUB, NSUB)])

    return k(vals, idx3d, zeros_n)


def _enc_node_body(pos_ref, typ_ref, emb_ref, w1_ref, b1_ref, w2_ref, b2_ref,
                   v_ref, tab0_ref):
    pos = pos_ref[...]
    nvel = pos[:, 3:18] - pos[:, 0:15]
    mr = pos[:, 15:18]
    dist = jnp.concatenate([mr - 0.1, 0.9 - mr], axis=1)
    distc = jnp.clip(dist * (1.0 / RADIUS), -1.0, 1.0)
    typ = typ_ref[...]
    oh = (typ == lax.broadcasted_iota(jnp.int32, (typ.shape[0], NUM_TYPES), 1))
    te = _dot(oh.astype(_f32), emb_ref[...])
    nf = jnp.concatenate([nvel, distc, te], axis=1)
    h = jnp.maximum(_dot(nf, w1_ref[...]) + b1_ref[...], 0.0)
    v = _ln(_dot(h, w2_ref[...]) + b2_ref[...])
    v_ref[...] = v
    tab0_ref[...] = jnp.concatenate(
        [v, mr, jnp.zeros((mr.shape[0], 61), _f32)], axis=1)


def _edge0_body(gs_ref, gr_ref, we1_ref, be1_ref, we2_ref, be2_ref,
                w1e_ref, w1s_ref, w1r_ref, b1_ref, w2_ref, b2_ref,
                enew_ref, eout_ref):
    vs = gs_ref[:, 0:LATENT]
    vr = gr_ref[:, 0:LATENT]
    rel = (gs_ref[:, LATENT:LATENT + 3]
           - gr_ref[:, LATENT:LATENT + 3]) * (1.0 / RADIUS)
    nrm = jnp.sqrt(jnp.sum(rel * rel, axis=1, keepdims=True))
    ef = jnp.concatenate([rel, nrm], axis=1)
    he = jnp.maximum(_dot(ef, we1_ref[...]) + be1_ref[...], 0.0)
    e = _ln(_dot(he, we2_ref[...]) + be2_ref[...])
    h = jnp.maximum(_dot(e, w1e_ref[...]) + _dot(vs, w1s_ref[...])
                    + _dot(vr, w1r_ref[...]) + b1_ref[...], 0.0)
    en = _ln(_dot(h, w2_ref[...]) + b2_ref[...])
    enew_ref[...] = jnp.concatenate(
        [en, jnp.zeros((en.shape[0], 128 - LATENT), _f32)], axis=1)
    eout_ref[...] = e + en


def _edge_step_body(e_ref, gs_ref, gr_ref, w1e_ref, b1_ref, w2_ref, b2_ref,
                    enew_ref, eout_ref):
    e = e_ref[...]
    h = jnp.maximum(_dot(e, w1e_ref[...]) + gs_ref[:, 0:LATENT]
                    + gr_ref[:, LATENT:] + b1_ref[...], 0.0)
    en = _ln(_dot(h, w2_ref[...]) + b2_ref[...])
    enew_ref[...] = jnp.concatenate(
        [en, jnp.zeros((en.shape[0], 128 - LATENT), _f32)], axis=1)
    eout_ref[...] = e + en


def _node_step_body(v_ref, parts_ref, partsb_ref, wv_ref, wa_ref, b1_ref,
                    w2_ref, b2_ref, ws_ref, wr_ref, vout_ref, p_ref):
    v = v_ref[...]
    agg = (parts_ref[0, :, 0:LATENT] + parts_ref[1, :, 0:LATENT]
           + partsb_ref[0, :, 0:LATENT] + partsb_ref[1, :, 0:LATENT])
    t = jnp.maximum(_dot(v, wv_ref[...]) + _dot(agg, wa_ref[...])
                    + b1_ref[...], 0.0)
    vo = v + _ln(_dot(t, w2_ref[...]) + b2_ref[...])
    vout_ref[...] = vo
    p_ref[...] = jnp.concatenate(
        [_dot(vo, ws_ref[...]), _dot(vo, wr_ref[...])], axis=1)


def _node_last_body(v_ref, parts_ref, partsb_ref, wv_ref, wa_ref, b1_ref,
                    w2_ref, b2_ref, vout_ref):
    v = v_ref[...]
    agg = (parts_ref[0, :, 0:LATENT] + parts_ref[1, :, 0:LATENT]
           + partsb_ref[0, :, 0:LATENT] + partsb_ref[1, :, 0:LATENT])
    t = jnp.maximum(_dot(v, wv_ref[...]) + _dot(agg, wa_ref[...])
                    + b1_ref[...], 0.0)
    vout_ref[...] = v + _ln(_dot(t, w2_ref[...]) + b2_ref[...])


def _dec_body(v_ref, pos_ref, w1_ref, b1_ref, w2_ref, b2_ref, out_ref):
    v = v_ref[...]
    t = jnp.maximum(_dot(v, w1_ref[...]) + b1_ref[...], 0.0)
    acc = _dot(t, w2_ref[...]) + b2_ref[...]
    pos = pos_ref[...]
    mr = pos[:, 15:18]
    pv = pos[:, 12:15]
    out3 = mr + (mr - pv) + acc[:, 0:3]
    out_ref[...] = jnp.concatenate([out3, acc[:, 3:8]], axis=1)


def _wspec(shape):
    return pl.BlockSpec(shape, lambda i: tuple(0 for _ in shape))


def _enc_node_call(pos18, typ, emb, w1, b1, w2, b2):
    return pl.pallas_call(
        _enc_node_body,
        grid=(NPAD // BN,),
        in_specs=[
            pl.BlockSpec((BN, 18), lambda i: (i, 0)),
            pl.BlockSpec((BN, 1), lambda i: (i, 0)),
            _wspec((NUM_TYPES, TYPE_EMB)),
            _wspec((37, LATENT)), _wspec((1, LATENT)),
            _wspec((LATENT, LATENT)), _wspec((1, LATENT)),
        ],
        out_specs=[
            pl.BlockSpec((BN, LATENT), lambda i: (i, 0)),
            pl.BlockSpec((BN, 128), lambda i: (i, 0)),
        ],
        out_shape=[
            jax.ShapeDtypeStruct((NPAD, LATENT), _f32),
            jax.ShapeDtypeStruct((NPAD, 128), _f32),
        ],
    )(pos18, typ, emb, w1, b1, w2, b2)


def _edge0_call(g, we1, be1, we2, be2, w1e, w1s, w1r, b1, w2, b2):
    return pl.pallas_call(
        _edge0_body,
        grid=(EH // BE,),
        in_specs=[
            pl.BlockSpec((BE, 128), lambda i: (i, 0)),
            pl.BlockSpec((BE, 128), lambda i: (i + EH // BE, 0)),
            _wspec((4, LATENT)), _wspec((1, LATENT)),
            _wspec((LATENT, LATENT)), _wspec((1, LATENT)),
            _wspec((LATENT, LATENT)), _wspec((LATENT, LATENT)),
            _wspec((LATENT, LATENT)), _wspec((1, LATENT)),
            _wspec((LATENT, LATENT)), _wspec((1, LATENT)),
        ],
        out_specs=[
            pl.BlockSpec((BE, 128), lambda i: (i, 0)),
            pl.BlockSpec((BE, LATENT), lambda i: (i, 0)),
        ],
        out_shape=[
            jax.ShapeDtypeStruct((EH, 128), _f32),
            jax.ShapeDtypeStruct((EH, LATENT), _f32),
        ],
    )(g, g, we1, be1, we2, be2, w1e, w1s, w1r, b1, w2, b2)


def _edge_step_call(e, g, w1e, b1, w2, b2):
    return pl.pallas_call(
        _edge_step_body,
        grid=(EH // BE,),
        in_specs=[
            pl.BlockSpec((BE, LATENT), lambda i: (i, 0)),
            pl.BlockSpec((BE, 128), lambda i: (i, 0)),
            pl.BlockSpec((BE, 128), lambda i: (i + EH // BE, 0)),
            _wspec((LATENT, LATENT)), _wspec((1, LATENT)),
            _wspec((LATENT, LATENT)), _wspec((1, LATENT)),
        ],
        out_specs=[
            pl.BlockSpec((BE, 128), lambda i: (i, 0)),
            pl.BlockSpec((BE, LATENT), lambda i: (i, 0)),
        ],
        out_shape=[
            jax.ShapeDtypeStruct((EH, 128), _f32),
            jax.ShapeDtypeStruct((EH, LATENT), _f32),
        ],
    )(e, g, g, w1e, b1, w2, b2)


def _node_step_call(v, parts, partsb, wv, wa, b1, w2, b2, ws, wr):
    return pl.pallas_call(
        _node_step_body,
        grid=(NPAD // BN,),
        in_specs=[
            pl.BlockSpec((BN, LATENT), lambda i: (i, 0)),
            pl.BlockSpec((2, BN, 128), lambda i: (0, i, 0)),
            pl.BlockSpec((2, BN, 128), lambda i: (0, i, 0)),
            _wspec((LATENT, LATENT)), _wspec((LATENT, LATENT)),
            _wspec((1, LATENT)),
            _wspec((LATENT, LATENT)), _wspec((1, LATENT)),
            _wspec((LATENT, LATENT)), _wspec((LATENT, LATENT)),
        ],
        out_specs=[
            pl.BlockSpec((BN, LATENT), lambda i: (i, 0)),
            pl.BlockSpec((BN, 128), lambda i: (i, 0)),
        ],
        out_shape=[
            jax.ShapeDtypeStruct((NPAD, LATENT), _f32),
            jax.ShapeDtypeStruct((NPAD, 128), _f32),
        ],
    )(v, parts, partsb, wv, wa, b1, w2, b2, ws, wr)


def _node_last_call(v, parts, partsb, wv, wa, b1, w2, b2):
    return pl.pallas_call(
        _node_last_body,
        grid=(NPAD // BN,),
        in_specs=[
            pl.BlockSpec((BN, LATENT), lambda i: (i, 0)),
            pl.BlockSpec((2, BN, 128), lambda i: (0, i, 0)),
            pl.BlockSpec((2, BN, 128), lambda i: (0, i, 0)),
            _wspec((LATENT, LATENT)), _wspec((LATENT, LATENT)),
            _wspec((1, LATENT)),
            _wspec((LATENT, LATENT)), _wspec((1, LATENT)),
        ],
        out_specs=pl.BlockSpec((BN, LATENT), lambda i: (i, 0)),
        out_shape=jax.ShapeDtypeStruct((NPAD, LATENT), _f32),
    )(v, parts, partsb, wv, wa, b1, w2, b2)


def _dec_call(v, pos18, w1, b1, w2, b2):
    return pl.pallas_call(
        _dec_body,
        grid=(NPAD // BN,),
        in_specs=[
            pl.BlockSpec((BN, LATENT), lambda i: (i, 0)),
            pl.BlockSpec((BN, 18), lambda i: (i, 0)),
            _wspec((LATENT, LATENT)), _wspec((1, LATENT)),
            _wspec((LATENT, 8)), _wspec((1, 8)),
        ],
        out_specs=pl.BlockSpec((BN, 8), lambda i: (i, 0)),
        out_shape=jax.ShapeDtypeStruct((NPAD, 8), _f32),
    )(v, pos18, w1, b1, w2, b2)


def kernel(position_sequence, params, particle_types, senders, receivers,
           n_particles_per_example):
    pos18 = jnp.pad(position_sequence.reshape(N, 18), ((0, NPAD - N), (0, 0)))
    typ = jnp.pad(particle_types.astype(jnp.int32), (0, NPAD - N))
    typ = typ.reshape(NPAD, 1)

    npad_e = EPAD - E
    pad_spread = (jnp.arange(npad_e, dtype=jnp.int32) * 37) % N
    s_pad = jnp.concatenate([senders.astype(jnp.int32), pad_spread])
    r_pad = jnp.concatenate([receivers.astype(jnp.int32), pad_spread])
    pad_sink = N + (jnp.arange(npad_e, dtype=jnp.int32) % (NPAD - N))
    r_sink = jnp.concatenate([receivers.astype(jnp.int32), pad_sink])
    idx_a = jnp.concatenate([s_pad[:EH], r_pad[:EH]]).reshape(-1, CHK)
    idx_b = jnp.concatenate([s_pad[EH:], r_pad[EH:]]).reshape(-1, CHK)
    ridx_a = r_sink[:EH].reshape(-1, CHK)
    ridx_b = r_sink[EH:].reshape(-1, CHK)
    zeros_n = jnp.zeros((NPAD, 128), _f32)

    def b2d(b):
        return b.reshape(1, -1)

    (wn1, bn1), (wn2, bn2) = params['enc_node']
    (we1, be1), (we2, be2) = params['enc_edge']
    (wd1, bd1), (wd2, bd2) = params['dec']
    wd2p = jnp.pad(wd2, ((0, 0), (0, 8 - DIMS)))
    bd2p = jnp.pad(bd2, (0, 8 - DIMS))

    ew1 = [sp['edge'][0][0] for sp in params['proc']]
    ws_all = [w[LATENT:2 * LATENT] for w in ew1]
    wr_all = [w[2 * LATENT:] for w in ew1]

    v, tab0 = _enc_node_call(pos18, typ, params['type_emb'],
                             wn1, b2d(bn1), wn2, b2d(bn2))

    p = tab0
    ea = eb = None
    for step in range(MP_STEPS):
        sp = params['proc'][step]
        (w1, b1), (w2, b2) = sp['edge']
        ga = _sc_gather(p, idx_a)
        gb = _sc_gather(p, idx_b)
        if step == 0:
            ena, ea = _edge0_call(ga, we1, b2d(be1), we2, b2d(be2),
                                  w1[:LATENT], ws_all[0], wr_all[0],
                                  b2d(b1), w2, b2d(b2))
            enb, eb = _edge0_call(gb, we1, b2d(be1), we2, b2d(be2),
                                  w1[:LATENT], ws_all[0], wr_all[0],
                                  b2d(b1), w2, b2d(b2))
        else:
            ena, ea = _edge_step_call(ea, ga, w1[:LATENT], b2d(b1), w2,
                                      b2d(b2))
            enb, eb = _edge_step_call(eb, gb, w1[:LATENT], b2d(b1), w2,
                                      b2d(b2))
        pa = _sc_scatter(ena, ridx_a, zeros_n)
        pb = _sc_scatter(enb, ridx_b, zeros_n)
        (nw1, nb1), (nw2, nb2) = sp['node']
        wv, wa = nw1[:LATENT], nw1[LATENT:]
        if step < MP_STEPS - 1:
            v, p = _node_step_call(v, pa, pb, wv, wa, b2d(nb1), nw2,
                                   b2d(nb2), ws_all[step + 1],
                                   wr_all[step + 1])
        else:
            v = _node_last_call(v, pa, pb, wv, wa, b2d(nb1), nw2, b2d(nb2))

    out = _dec_call(v, pos18, wd1, b2d(bd1), wd2p, b2d(bd2p))
    return out[:N, :DIMS]
